# Initial kernel scaffold; baseline (speedup 1.0000x reference)
import jax
import jax.numpy as jnp
from jax.experimental import pallas as pl


def kernel(x, edge_index, edge_weight, c0_pre_h_W, c0_pre_h_b, c0_pre_x_W, c0_pre_x_b, c0_cheb_lin0_W, c0_cheb_lin1_W, c0_cheb_b, c0_arma_init_W, c0_arma_root_W, c0_arma_b, c0_lin_W, c0_lin_b, c1_pre_h_W, c1_pre_h_b, c1_pre_x_W, c1_pre_x_b, c1_cheb_lin0_W, c1_cheb_lin1_W, c1_cheb_b, c1_arma_init_W, c1_arma_root_W, c1_arma_b, c1_lin_W, c1_lin_b, cls_W, cls_b):
    raise NotImplementedError("write your pallas kernel here")


# speedup vs baseline: 24.1372x; 24.1372x over previous
import jax
import jax.numpy as jnp
from jax import lax
from jax.experimental import pallas as pl
from jax.experimental.pallas import tpu as pltpu
from jax.experimental.pallas import tpu_sc as plsc

N = 10000
NC_CLS = 32
E = 320000

NCORES = 2
NSUB = 16
NW = NCORES * NSUB

SUB = 128
NSC = 2
CH = SUB * NSC
GRP = CH // 16

EP = 327680
PAD = EP - E
E_DEG = EP // NSUB
E_PASS = EP // NW
DEG_CHUNKS = E_DEG // CH
PASS_CHUNKS = E_PASS // CH

DN = 10240
DROWS = DN // NSUB


def _newton_rsqrt(v):
  b = lax.bitcast_convert_type(v, jnp.int32)
  h = jnp.int32(0x5F3759DF) - (b >> 1)
  y = lax.bitcast_convert_type(h, jnp.float32)
  for _ in range(4):
    y = y * (1.5 - 0.5 * v * y * y)
  return y


def _zero_vmem2d(ref, rows, cols):
  z = jnp.zeros((16,), jnp.float32)
  for r in range(rows):
    for c0 in range(cols // 16):
      ref[r, pl.ds(c0 * 16, 16)] = z


def _zero_acc(acc_sh, zrow, s):
  _zero_vmem2d(zrow, 16, 128)
  rbase = s * DROWS
  for j in range(DROWS // 16):
    pltpu.sync_copy(zrow, acc_sh.at[pl.ds(rbase + j * 16, 16), :])


def _drain(descs):
  for d in descs:
    d.wait()


def _scale_rows(rows_ref, nc_ref, na_ref):
  @pl.loop(0, GRP)
  def _(g):
    base = pl.multiple_of(g * 16, 16)
    ncv = nc_ref[pl.ds(base, 16)]
    nav = na_ref[pl.ds(base, 16)]
    for i in range(16):
      e = base + i
      ncs = ncv[i]
      nas = nav[i]
      for f in range(4):
        rows_ref[e, pl.ds(f * 16, 16)] = rows_ref[e, pl.ds(f * 16, 16)] * ncs
      for f in range(4, 8):
        rows_ref[e, pl.ds(f * 16, 16)] = rows_ref[e, pl.ds(f * 16, 16)] * nas


def _load_idx_chunk(src_hbm, dst_hbm, off, i2s, i2d, sem):
  descs = []
  for k in range(NSC):
    descs.append(pltpu.async_copy(
        src_hbm.at[pl.ds(off + k * SUB, SUB)], i2s[k], sem))
    descs.append(pltpu.async_copy(
        dst_hbm.at[pl.ds(off + k * SUB, SUB)], i2d[k], sem))
  return descs


def _make_sc_pass1():
  mesh = plsc.VectorSubcoreMesh(
      core_axis_name="c", subcore_axis_name="s",
      num_cores=NCORES, num_subcores=NSUB)

  def body(src_hbm, dst_hbm, ew_hbm, g_hbm, p_hbm, normc_hbm, norma_hbm,
           acc_sh, degc_sh, dega_sh,
           zrow,
           i2s0, i2s1, i2d0, i2d1,
           w_b, wc_b, nc_b, na_b, dcs, dcd, das, dad, dwork, rows,
           sem_in, sem_g, sem_out):
    c = lax.axis_index("c")
    s = lax.axis_index("s")
    i2s = [i2s0, i2s1]
    i2d = [i2d0, i2d1]

    _zero_acc(acc_sh, zrow, s)
    zflat = zrow.at[0]
    @pl.when(s == 0)
    def _():
      for j in range(DN // 128):
        pltpu.sync_copy(zflat, degc_sh.at[pl.ds(j * 128, 128)])
    @pl.when(s == 1)
    def _():
      for j in range(DN // 128):
        pltpu.sync_copy(zflat, dega_sh.at[pl.ds(j * 128, 128)])
    plsc.subcore_barrier()

    dbase = s * E_DEG

    dbase = s * E_DEG

    @pl.loop(0, DEG_CHUNKS)
    def _(j):
      off = pl.multiple_of(dbase + j * CH, CH)
      descs = _load_idx_chunk(src_hbm, dst_hbm, off, i2s, i2d, sem_in)
      descs.append(pltpu.async_copy(ew_hbm.at[pl.ds(off, CH)], w_b, sem_in))
      _drain(descs)
      for g in range(GRP):
        sl = pl.ds(g * 16, 16)
        k, col = divmod(g * 16, SUB)
        sv = i2s[k][pl.ds(col, 16)]
        dv = i2d[k][pl.ds(col, 16)]
        wc_b[sl] = jnp.where(sv == dv, 0.0, w_b[sl])
      descs = []
      for k in range(NSC):
        descs.append(pltpu.async_copy(
            wc_b.at[pl.ds(k * SUB, SUB)], degc_sh.at[i2s[k]],
            sem_out, add=True))
        descs.append(pltpu.async_copy(
            w_b.at[pl.ds(k * SUB, SUB)], dega_sh.at[i2d[k]],
            sem_out, add=True))
      _drain(descs)

    plsc.subcore_barrier()

    rbase = s * DROWS
    for deg_sh in (degc_sh, dega_sh):
      pltpu.sync_copy(deg_sh.at[pl.ds(rbase, DROWS)], dwork)
      @pl.loop(0, DROWS // 16)
      def _(i):
        sl = pl.ds(pl.multiple_of(i * 16, 16), 16)
        v = dwork[sl]
        dwork[sl] = jnp.where(v > 0, _newton_rsqrt(v), 0.0)
      pltpu.sync_copy(dwork, deg_sh.at[pl.ds(rbase, DROWS)])
    plsc.subcore_barrier()

    ebase = (c * NSUB + s) * E_PASS

    @pl.loop(0, PASS_CHUNKS)
    def _(j):
      off = pl.multiple_of(ebase + j * CH, CH)
      descs = _load_idx_chunk(src_hbm, dst_hbm, off, i2s, i2d, sem_in)
      descs.append(pltpu.async_copy(ew_hbm.at[pl.ds(off, CH)], w_b, sem_in))
      _drain(descs)
      descs = []
      for k in range(NSC):
        sl = pl.ds(k * SUB, SUB)
        descs.append(pltpu.async_copy(degc_sh.at[i2s[k]], dcs.at[sl], sem_g))
        descs.append(pltpu.async_copy(degc_sh.at[i2d[k]], dcd.at[sl], sem_g))
        descs.append(pltpu.async_copy(dega_sh.at[i2s[k]], das.at[sl], sem_g))
        descs.append(pltpu.async_copy(dega_sh.at[i2d[k]], dad.at[sl], sem_g))
      _drain(descs)
      descs = []
      for k in range(NSC):
        descs.append(pltpu.async_copy(
            g_hbm.at[i2s[k]], rows.at[pl.ds(k * SUB, SUB), :], sem_in))
      _drain(descs)
      for g in range(GRP):
        sl = pl.ds(g * 16, 16)
        k, col = divmod(g * 16, SUB)
        sv = i2s[k][pl.ds(col, 16)]
        dv = i2d[k][pl.ds(col, 16)]
        wv = w_b[sl]
        wc = jnp.where(sv == dv, 0.0, wv)
        nc_b[sl] = -(dcs[sl] * wc * dcd[sl])
        na_b[sl] = das[sl] * wv * dad[sl]
      _scale_rows(rows, nc_b, na_b)
      pltpu.sync_copy(nc_b, normc_hbm.at[pl.ds(off, CH)])
      pltpu.sync_copy(na_b, norma_hbm.at[pl.ds(off, CH)])
      descs = []
      for k in range(NSC):
        descs.append(pltpu.async_copy(
            rows.at[pl.ds(k * SUB, SUB), :], acc_sh.at[i2d[k]],
            sem_out, add=True))
      _drain(descs)

    plsc.subcore_barrier()

    @pl.when(s < NSUB - 1)
    def _():
      pltpu.sync_copy(acc_sh.at[pl.ds(rbase, DROWS), :],
                      p_hbm.at[c, pl.ds(rbase, DROWS), :])
    @pl.when(s == NSUB - 1)
    def _():
      pltpu.sync_copy(acc_sh.at[pl.ds(rbase, N - (NSUB - 1) * DROWS), :],
                      p_hbm.at[c, pl.ds(rbase, N - (NSUB - 1) * DROWS), :])

  return pl.kernel(
      body,
      out_type=(
          jax.ShapeDtypeStruct((NCORES, N, 128), jnp.float32),
          jax.ShapeDtypeStruct((EP,), jnp.float32),
          jax.ShapeDtypeStruct((EP,), jnp.float32),
      ),
      mesh=mesh,
      compiler_params=pltpu.CompilerParams(use_tc_tiling_on_sc=False),
      scratch_types=[
          pltpu.VMEM_SHARED((DN, 128), jnp.float32),
          pltpu.VMEM_SHARED((DN,), jnp.float32),
          pltpu.VMEM_SHARED((DN,), jnp.float32),
          pltpu.VMEM((16, 128), jnp.float32),
          pltpu.VMEM((SUB,), jnp.int32),
          pltpu.VMEM((SUB,), jnp.int32),
          pltpu.VMEM((SUB,), jnp.int32),
          pltpu.VMEM((SUB,), jnp.int32),
          pltpu.VMEM((CH,), jnp.float32),
          pltpu.VMEM((CH,), jnp.float32),
          pltpu.VMEM((CH,), jnp.float32),
          pltpu.VMEM((CH,), jnp.float32),
          pltpu.VMEM((CH,), jnp.float32),
          pltpu.VMEM((CH,), jnp.float32),
          pltpu.VMEM((CH,), jnp.float32),
          pltpu.VMEM((CH,), jnp.float32),
          pltpu.VMEM((DROWS,), jnp.float32),
          pltpu.VMEM((CH, 128), jnp.float32),
          pltpu.SemaphoreType.DMA,
          pltpu.SemaphoreType.DMA,
          pltpu.SemaphoreType.DMA,
      ],
      name="sc_deg_norm_pass0",
  )


def _make_sc_pass2():
  mesh = plsc.VectorSubcoreMesh(
      core_axis_name="c", subcore_axis_name="s",
      num_cores=NCORES, num_subcores=NSUB)

  def body(src_hbm, dst_hbm, normc_hbm, norma_hbm, g_hbm, p_hbm,
           acc_sh, zrow,
           i2s0, i2s1, i2d0, i2d1,
           nc_b, na_b, rows,
           sem_in, sem_g, sem_out):
    c = lax.axis_index("c")
    s = lax.axis_index("s")
    i2s = [i2s0, i2s1]
    i2d = [i2d0, i2d1]

    _zero_acc(acc_sh, zrow, s)
    plsc.subcore_barrier()

    ebase = (c * NSUB + s) * E_PASS

    @pl.loop(0, PASS_CHUNKS)
    def _(j):
      off = pl.multiple_of(ebase + j * CH, CH)
      descs = _load_idx_chunk(src_hbm, dst_hbm, off, i2s, i2d, sem_in)
      descs.append(pltpu.async_copy(
          normc_hbm.at[pl.ds(off, CH)], nc_b, sem_in))
      descs.append(pltpu.async_copy(
          norma_hbm.at[pl.ds(off, CH)], na_b, sem_in))
      _drain(descs)
      descs = []
      for k in range(NSC):
        descs.append(pltpu.async_copy(
            g_hbm.at[i2s[k]], rows.at[pl.ds(k * SUB, SUB), :], sem_g))
      _drain(descs)
      _scale_rows(rows, nc_b, na_b)
      descs = []
      for k in range(NSC):
        descs.append(pltpu.async_copy(
            rows.at[pl.ds(k * SUB, SUB), :], acc_sh.at[i2d[k]],
            sem_out, add=True))
      _drain(descs)

    plsc.subcore_barrier()
    rbase = s * DROWS
    @pl.when(s < NSUB - 1)
    def _():
      pltpu.sync_copy(acc_sh.at[pl.ds(rbase, DROWS), :],
                      p_hbm.at[c, pl.ds(rbase, DROWS), :])
    @pl.when(s == NSUB - 1)
    def _():
      pltpu.sync_copy(acc_sh.at[pl.ds(rbase, N - (NSUB - 1) * DROWS), :],
                      p_hbm.at[c, pl.ds(rbase, N - (NSUB - 1) * DROWS), :])

  return pl.kernel(
      body,
      out_type=jax.ShapeDtypeStruct((NCORES, N, 128), jnp.float32),
      mesh=mesh,
      compiler_params=pltpu.CompilerParams(use_tc_tiling_on_sc=False),
      scratch_types=[
          pltpu.VMEM_SHARED((DN, 128), jnp.float32),
          pltpu.VMEM((16, 128), jnp.float32),
          pltpu.VMEM((SUB,), jnp.int32),
          pltpu.VMEM((SUB,), jnp.int32),
          pltpu.VMEM((SUB,), jnp.int32),
          pltpu.VMEM((SUB,), jnp.int32),
          pltpu.VMEM((CH,), jnp.float32),
          pltpu.VMEM((CH,), jnp.float32),
          pltpu.VMEM((CH, 128), jnp.float32),
          pltpu.SemaphoreType.DMA,
          pltpu.SemaphoreType.DMA,
          pltpu.SemaphoreType.DMA,
      ],
      name="sc_pass1",
  )


_sc_pass1 = _make_sc_pass1()
_sc_pass2 = _make_sc_pass2()


RB = 1000
GRID = N // RB


def _dotT(a, w):
  return lax.dot_general(a, w, (((1,), (1,)), ((), ())),
                         preferred_element_type=jnp.float32)


def _dot(a, w):
  return lax.dot_general(a, w, (((1,), (0,)), ((), ())),
                         preferred_element_type=jnp.float32)


def _tc1_body(x_r, pxW_r, pxb_r, phW_r, phb_r, aiW_r, g_r, xp_r):
  xb = x_r[...]
  xp = _dotT(xb, pxW_r[...]) + pxb_r[...]
  hp = _dotT(xb, phW_r[...]) + phb_r[...]
  hh = _dot(xp, aiW_r[...])
  g_r[...] = jnp.concatenate([hp, hh], axis=1)
  xp_r[...] = xp


def _tc2_body(x_r, g0_r, xp0_r, p_r,
              cl0_r, cl1_r, cb_r, arW_r, ab_r, lW_r, lb_r,
              pxW_r, pxb_r, phW_r, phb_r, aiW_r,
              g1_r, xp1_r):
  p = p_r[0] + p_r[1]
  tx1 = p[:, :64]
  agg = p[:, 64:]
  hp0 = g0_r[:, :64]
  xp0 = xp0_r[...]
  o1 = _dotT(hp0, cl0_r[...]) + _dotT(tx1, cl1_r[...]) + cb_r[...]
  o1 = jnp.where(o1 >= 0, o1, 0.01 * o1)
  o2 = agg + _dot(xp0, arW_r[...]) + ab_r[...]
  o2 = jnp.maximum(o2, 0.0)
  o3 = _dotT(o1 + o2, lW_r[...]) + lb_r[...]
  xp1 = _dotT(o3, pxW_r[...]) + pxb_r[...]
  hp1 = _dotT(x_r[...], phW_r[...]) + phb_r[...]
  hh1 = _dot(xp1, aiW_r[...])
  g1_r[...] = jnp.concatenate([hp1, hh1], axis=1)
  xp1_r[...] = xp1


def _tc3_body(g1_r, xp1_r, p_r,
              cl0_r, cl1_r, cb_r, arW_r, ab_r, lW_r, lb_r,
              clsW_r, clsb_r, out_r):
  p = p_r[0] + p_r[1]
  tx1 = p[:, :64]
  agg = p[:, 64:]
  hp1 = g1_r[:, :64]
  o1 = _dotT(hp1, cl0_r[...]) + _dotT(tx1, cl1_r[...]) + cb_r[...]
  o1 = jnp.where(o1 >= 0, o1, 0.01 * o1)
  o2 = agg + _dot(xp1_r[...], arW_r[...]) + ab_r[...]
  o2 = jnp.maximum(o2, 0.0)
  o3 = _dotT(o1 + o2, lW_r[...]) + lb_r[...]
  logits = _dotT(o3, clsW_r[...]) + clsb_r[...]
  m = jnp.max(logits, axis=1, keepdims=True)
  sh = logits - m
  out_r[...] = sh - jnp.log(jnp.sum(jnp.exp(sh), axis=1, keepdims=True))


def _full(shape):
  return pl.BlockSpec(shape, lambda i: (0,) * len(shape))


def _rows(shape):
  return pl.BlockSpec(shape, lambda i: (i,) + (0,) * (len(shape) - 1))


def kernel(x, edge_index, edge_weight,
           c0_pre_h_W, c0_pre_h_b, c0_pre_x_W, c0_pre_x_b,
           c0_cheb_lin0_W, c0_cheb_lin1_W, c0_cheb_b,
           c0_arma_init_W, c0_arma_root_W, c0_arma_b,
           c0_lin_W, c0_lin_b,
           c1_pre_h_W, c1_pre_h_b, c1_pre_x_W, c1_pre_x_b,
           c1_cheb_lin0_W, c1_cheb_lin1_W, c1_cheb_b,
           c1_arma_init_W, c1_arma_root_W, c1_arma_b,
           c1_lin_W, c1_lin_b,
           cls_W, cls_b):
  r2 = lambda b: b.reshape(1, -1)

  pad_idx = (jnp.arange(PAD, dtype=jnp.int32) * 997) % N
  src = jnp.concatenate([edge_index[0], pad_idx])
  dst = jnp.concatenate([edge_index[1], pad_idx])
  ew = jnp.concatenate([edge_weight, jnp.zeros((PAD,), jnp.float32)])

  g0, xp0 = pl.pallas_call(
      _tc1_body,
      grid=(GRID,),
      in_specs=[_rows((RB, 128)), _full((64, 128)), _full((1, 64)),
                _full((64, 128)), _full((1, 64)), _full((64, 64))],
      out_specs=[_rows((RB, 128)), _rows((RB, 64))],
      out_shape=[jax.ShapeDtypeStruct((N, 128), jnp.float32),
                 jax.ShapeDtypeStruct((N, 64), jnp.float32)],
  )(x, c0_pre_x_W, r2(c0_pre_x_b), c0_pre_h_W, r2(c0_pre_h_b),
    c0_arma_init_W)

  p0, normc, norma = _sc_pass1(src, dst, ew, g0)

  g1, xp1 = pl.pallas_call(
      _tc2_body,
      grid=(GRID,),
      in_specs=[_rows((RB, 128)), _rows((RB, 128)), _rows((RB, 64)),
                pl.BlockSpec((2, RB, 128), lambda i: (0, i, 0)),
                _full((64, 64)), _full((64, 64)), _full((1, 64)),
                _full((64, 64)), _full((1, 64)),
                _full((64, 64)), _full((1, 64)),
                _full((64, 64)), _full((1, 64)),
                _full((64, 128)), _full((1, 64)), _full((64, 64))],
      out_specs=[_rows((RB, 128)), _rows((RB, 64))],
      out_shape=[jax.ShapeDtypeStruct((N, 128), jnp.float32),
                 jax.ShapeDtypeStruct((N, 64), jnp.float32)],
  )(x, g0, xp0, p0,
    c0_cheb_lin0_W, c0_cheb_lin1_W, r2(c0_cheb_b),
    c0_arma_root_W, r2(c0_arma_b), c0_lin_W, r2(c0_lin_b),
    c1_pre_x_W, r2(c1_pre_x_b), c1_pre_h_W, r2(c1_pre_h_b),
    c1_arma_init_W)

  p1 = _sc_pass2(src, dst, normc, norma, g1)

  out = pl.pallas_call(
      _tc3_body,
      grid=(GRID,),
      in_specs=[_rows((RB, 128)), _rows((RB, 64)),
                pl.BlockSpec((2, RB, 128), lambda i: (0, i, 0)),
                _full((64, 64)), _full((64, 64)), _full((1, 64)),
                _full((64, 64)), _full((1, 64)),
                _full((64, 64)), _full((1, 64)),
                _full((32, 64)), _full((1, 32))],
      out_specs=_rows((RB, NC_CLS)),
      out_shape=jax.ShapeDtypeStruct((N, NC_CLS), jnp.float32),
  )(g1, xp1, p1,
    c1_cheb_lin0_W, c1_cheb_lin1_W, r2(c1_cheb_b),
    c1_arma_root_W, r2(c1_arma_b), c1_lin_W, r2(c1_lin_b),
    cls_W, r2(cls_b))

  return out


# speedup vs baseline: 25.3294x; 1.0494x over previous
import jax
import jax.numpy as jnp
from jax import lax
from jax.experimental import pallas as pl
from jax.experimental.pallas import tpu as pltpu
from jax.experimental.pallas import tpu_sc as plsc

N = 10000
NC_CLS = 32
E = 320000

NCORES = 2
NSUB = 16
NW = NCORES * NSUB

SUB = 128
NSC = 2
CH = SUB * NSC
GRP = CH // 16

EP = 327680
PAD = EP - E
E_DEG = EP // NSUB
E_PASS = EP // NW
DEG_CHUNKS = E_DEG // CH
PASS_CHUNKS = E_PASS // CH

DN = 10240
DROWS = DN // NSUB


def _newton_rsqrt(v):
  b = lax.bitcast_convert_type(v, jnp.int32)
  h = jnp.int32(0x5F3759DF) - (b >> 1)
  y = lax.bitcast_convert_type(h, jnp.float32)
  for _ in range(4):
    y = y * (1.5 - 0.5 * v * y * y)
  return y


def _zero_vmem2d(ref, rows, cols):
  z = jnp.zeros((16,), jnp.float32)
  for r in range(rows):
    for c0 in range(cols // 16):
      ref[r, pl.ds(c0 * 16, 16)] = z


def _zero_acc(acc_sh, zrow, s, sem):
  _zero_vmem2d(zrow, 16, 128)
  rbase = s * DROWS
  descs = []
  for j in range(DROWS // 16):
    descs.append(pltpu.async_copy(
        zrow, acc_sh.at[pl.ds(rbase + j * 16, 16), :], sem))
  _drain(descs)


def _drain(descs):
  for d in descs:
    d.wait()


def _scale_rows(rows_ref, nc_ref, na_ref):
  @pl.loop(0, GRP)
  def _(g):
    base = pl.multiple_of(g * 16, 16)
    ncv = nc_ref[pl.ds(base, 16)]
    nav = na_ref[pl.ds(base, 16)]
    for i in range(16):
      e = base + i
      ncs = ncv[i]
      nas = nav[i]
      for f in range(4):
        rows_ref[e, pl.ds(f * 16, 16)] = rows_ref[e, pl.ds(f * 16, 16)] * ncs
      for f in range(4, 8):
        rows_ref[e, pl.ds(f * 16, 16)] = rows_ref[e, pl.ds(f * 16, 16)] * nas


def _load_idx_chunk(src_hbm, dst_hbm, off, i2s, i2d, sem):
  descs = []
  for k in range(NSC):
    descs.append(pltpu.async_copy(
        src_hbm.at[pl.ds(off + k * SUB, SUB)], i2s[k], sem))
    descs.append(pltpu.async_copy(
        dst_hbm.at[pl.ds(off + k * SUB, SUB)], i2d[k], sem))
  return descs


def _make_sc_pass1():
  mesh = plsc.VectorSubcoreMesh(
      core_axis_name="c", subcore_axis_name="s",
      num_cores=NCORES, num_subcores=NSUB)

  def body(src_hbm, dst_hbm, ew_hbm, g_hbm, p_hbm, normc_hbm, norma_hbm,
           acc_sh, degc_sh, dega_sh,
           zrow,
           i2s0, i2s1, i2d0, i2d1,
           w_b, wc_b, nc_b, na_b, dcs, dcd, das, dad, dwork, rows,
           sem_in, sem_g, sem_out, sem_rows):
    c = lax.axis_index("c")
    s = lax.axis_index("s")
    i2s = [i2s0, i2s1]
    i2d = [i2d0, i2d1]

    _zero_acc(acc_sh, zrow, s, sem_in)
    zflat = zrow.at[0]
    descs = []
    for j in range(DROWS // 128):
      descs.append(pltpu.async_copy(
          zflat, degc_sh.at[pl.ds(s * DROWS + j * 128, 128)], sem_in))
      descs.append(pltpu.async_copy(
          zflat, dega_sh.at[pl.ds(s * DROWS + j * 128, 128)], sem_in))
    _drain(descs)
    plsc.subcore_barrier()

    dbase = s * E_DEG

    dbase = s * E_DEG

    @pl.loop(0, DEG_CHUNKS)
    def _(j):
      off = pl.multiple_of(dbase + j * CH, CH)
      descs = _load_idx_chunk(src_hbm, dst_hbm, off, i2s, i2d, sem_in)
      descs.append(pltpu.async_copy(ew_hbm.at[pl.ds(off, CH)], w_b, sem_in))
      _drain(descs)
      for g in range(GRP):
        sl = pl.ds(g * 16, 16)
        k, col = divmod(g * 16, SUB)
        sv = i2s[k][pl.ds(col, 16)]
        dv = i2d[k][pl.ds(col, 16)]
        wc_b[sl] = jnp.where(sv == dv, 0.0, w_b[sl])
      descs = []
      for k in range(NSC):
        descs.append(pltpu.async_copy(
            wc_b.at[pl.ds(k * SUB, SUB)], degc_sh.at[i2s[k]],
            sem_out, add=True))
        descs.append(pltpu.async_copy(
            w_b.at[pl.ds(k * SUB, SUB)], dega_sh.at[i2d[k]],
            sem_out, add=True))
      _drain(descs)

    plsc.subcore_barrier()

    rbase = s * DROWS
    for deg_sh in (degc_sh, dega_sh):
      pltpu.sync_copy(deg_sh.at[pl.ds(rbase, DROWS)], dwork)
      @pl.loop(0, DROWS // 16)
      def _(i):
        sl = pl.ds(pl.multiple_of(i * 16, 16), 16)
        v = dwork[sl]
        dwork[sl] = jnp.where(v > 0, _newton_rsqrt(v), 0.0)
      pltpu.sync_copy(dwork, deg_sh.at[pl.ds(rbase, DROWS)])
    plsc.subcore_barrier()

    ebase = (c * NSUB + s) * E_PASS

    @pl.loop(0, PASS_CHUNKS)
    def _(j):
      off = pl.multiple_of(ebase + j * CH, CH)
      descs = _load_idx_chunk(src_hbm, dst_hbm, off, i2s, i2d, sem_in)
      descs.append(pltpu.async_copy(ew_hbm.at[pl.ds(off, CH)], w_b, sem_in))
      _drain(descs)
      descs = []
      for k in range(NSC):
        sl = pl.ds(k * SUB, SUB)
        descs.append(pltpu.async_copy(degc_sh.at[i2s[k]], dcs.at[sl], sem_g))
        descs.append(pltpu.async_copy(degc_sh.at[i2d[k]], dcd.at[sl], sem_g))
        descs.append(pltpu.async_copy(dega_sh.at[i2s[k]], das.at[sl], sem_g))
        descs.append(pltpu.async_copy(dega_sh.at[i2d[k]], dad.at[sl], sem_g))
      rdescs = []
      for k in range(NSC):
        rdescs.append(pltpu.async_copy(
            g_hbm.at[i2s[k]], rows.at[pl.ds(k * SUB, SUB), :], sem_rows))
      _drain(descs)
      for g in range(GRP):
        sl = pl.ds(g * 16, 16)
        k, col = divmod(g * 16, SUB)
        sv = i2s[k][pl.ds(col, 16)]
        dv = i2d[k][pl.ds(col, 16)]
        wv = w_b[sl]
        wc = jnp.where(sv == dv, 0.0, wv)
        nc_b[sl] = -(dcs[sl] * wc * dcd[sl])
        na_b[sl] = das[sl] * wv * dad[sl]
      ndescs = [
          pltpu.async_copy(nc_b, normc_hbm.at[pl.ds(off, CH)], sem_in),
          pltpu.async_copy(na_b, norma_hbm.at[pl.ds(off, CH)], sem_in),
      ]
      _drain(rdescs)
      _scale_rows(rows, nc_b, na_b)
      descs = []
      for k in range(NSC):
        descs.append(pltpu.async_copy(
            rows.at[pl.ds(k * SUB, SUB), :], acc_sh.at[i2d[k]],
            sem_out, add=True))
      _drain(ndescs)
      _drain(descs)

    plsc.subcore_barrier()

    @pl.when(s < NSUB - 1)
    def _():
      pltpu.sync_copy(acc_sh.at[pl.ds(rbase, DROWS), :],
                      p_hbm.at[c, pl.ds(rbase, DROWS), :])
    @pl.when(s == NSUB - 1)
    def _():
      pltpu.sync_copy(acc_sh.at[pl.ds(rbase, N - (NSUB - 1) * DROWS), :],
                      p_hbm.at[c, pl.ds(rbase, N - (NSUB - 1) * DROWS), :])

  return pl.kernel(
      body,
      out_type=(
          jax.ShapeDtypeStruct((NCORES, N, 128), jnp.float32),
          jax.ShapeDtypeStruct((EP,), jnp.float32),
          jax.ShapeDtypeStruct((EP,), jnp.float32),
      ),
      mesh=mesh,
      compiler_params=pltpu.CompilerParams(use_tc_tiling_on_sc=False),
      scratch_types=[
          pltpu.VMEM_SHARED((DN, 128), jnp.float32),
          pltpu.VMEM_SHARED((DN,), jnp.float32),
          pltpu.VMEM_SHARED((DN,), jnp.float32),
          pltpu.VMEM((16, 128), jnp.float32),
          pltpu.VMEM((SUB,), jnp.int32),
          pltpu.VMEM((SUB,), jnp.int32),
          pltpu.VMEM((SUB,), jnp.int32),
          pltpu.VMEM((SUB,), jnp.int32),
          pltpu.VMEM((CH,), jnp.float32),
          pltpu.VMEM((CH,), jnp.float32),
          pltpu.VMEM((CH,), jnp.float32),
          pltpu.VMEM((CH,), jnp.float32),
          pltpu.VMEM((CH,), jnp.float32),
          pltpu.VMEM((CH,), jnp.float32),
          pltpu.VMEM((CH,), jnp.float32),
          pltpu.VMEM((CH,), jnp.float32),
          pltpu.VMEM((DROWS,), jnp.float32),
          pltpu.VMEM((CH, 128), jnp.float32),
          pltpu.SemaphoreType.DMA,
          pltpu.SemaphoreType.DMA,
          pltpu.SemaphoreType.DMA,
          pltpu.SemaphoreType.DMA,
      ],
      name="sc_deg_norm_pass0",
  )


def _make_sc_pass2():
  mesh = plsc.VectorSubcoreMesh(
      core_axis_name="c", subcore_axis_name="s",
      num_cores=NCORES, num_subcores=NSUB)

  def body(src_hbm, dst_hbm, normc_hbm, norma_hbm, g_hbm, p_hbm,
           acc_sh, zrow,
           i2s0, i2s1, i2d0, i2d1,
           nc_b, na_b, rows,
           sem_in, sem_g, sem_out):
    c = lax.axis_index("c")
    s = lax.axis_index("s")
    i2s = [i2s0, i2s1]
    i2d = [i2d0, i2d1]

    _zero_acc(acc_sh, zrow, s, sem_in)
    plsc.subcore_barrier()

    ebase = (c * NSUB + s) * E_PASS

    @pl.loop(0, PASS_CHUNKS)
    def _(j):
      off = pl.multiple_of(ebase + j * CH, CH)
      descs = _load_idx_chunk(src_hbm, dst_hbm, off, i2s, i2d, sem_in)
      descs.append(pltpu.async_copy(
          normc_hbm.at[pl.ds(off, CH)], nc_b, sem_in))
      descs.append(pltpu.async_copy(
          norma_hbm.at[pl.ds(off, CH)], na_b, sem_in))
      _drain(descs)
      descs = []
      for k in range(NSC):
        descs.append(pltpu.async_copy(
            g_hbm.at[i2s[k]], rows.at[pl.ds(k * SUB, SUB), :], sem_g))
      _drain(descs)
      _scale_rows(rows, nc_b, na_b)
      descs = []
      for k in range(NSC):
        descs.append(pltpu.async_copy(
            rows.at[pl.ds(k * SUB, SUB), :], acc_sh.at[i2d[k]],
            sem_out, add=True))
      _drain(descs)

    plsc.subcore_barrier()
    rbase = s * DROWS
    @pl.when(s < NSUB - 1)
    def _():
      pltpu.sync_copy(acc_sh.at[pl.ds(rbase, DROWS), :],
                      p_hbm.at[c, pl.ds(rbase, DROWS), :])
    @pl.when(s == NSUB - 1)
    def _():
      pltpu.sync_copy(acc_sh.at[pl.ds(rbase, N - (NSUB - 1) * DROWS), :],
                      p_hbm.at[c, pl.ds(rbase, N - (NSUB - 1) * DROWS), :])

  return pl.kernel(
      body,
      out_type=jax.ShapeDtypeStruct((NCORES, N, 128), jnp.float32),
      mesh=mesh,
      compiler_params=pltpu.CompilerParams(use_tc_tiling_on_sc=False),
      scratch_types=[
          pltpu.VMEM_SHARED((DN, 128), jnp.float32),
          pltpu.VMEM((16, 128), jnp.float32),
          pltpu.VMEM((SUB,), jnp.int32),
          pltpu.VMEM((SUB,), jnp.int32),
          pltpu.VMEM((SUB,), jnp.int32),
          pltpu.VMEM((SUB,), jnp.int32),
          pltpu.VMEM((CH,), jnp.float32),
          pltpu.VMEM((CH,), jnp.float32),
          pltpu.VMEM((CH, 128), jnp.float32),
          pltpu.SemaphoreType.DMA,
          pltpu.SemaphoreType.DMA,
          pltpu.SemaphoreType.DMA,
      ],
      name="sc_pass1",
  )


_sc_pass1 = _make_sc_pass1()
_sc_pass2 = _make_sc_pass2()


RB = 1000
GRID = N // RB


def _dotT(a, w):
  return lax.dot_general(a, w, (((1,), (1,)), ((), ())),
                         preferred_element_type=jnp.float32)


def _dot(a, w):
  return lax.dot_general(a, w, (((1,), (0,)), ((), ())),
                         preferred_element_type=jnp.float32)


def _tc1_body(x_r, pxW_r, pxb_r, phW_r, phb_r, aiW_r, g_r, xp_r):
  xb = x_r[...]
  xp = _dotT(xb, pxW_r[...]) + pxb_r[...]
  hp = _dotT(xb, phW_r[...]) + phb_r[...]
  hh = _dot(xp, aiW_r[...])
  g_r[...] = jnp.concatenate([hp, hh], axis=1)
  xp_r[...] = xp


def _tc2_body(x_r, g0_r, xp0_r, p_r,
              cl0_r, cl1_r, cb_r, arW_r, ab_r, lW_r, lb_r,
              pxW_r, pxb_r, phW_r, phb_r, aiW_r,
              g1_r, xp1_r):
  p = p_r[0] + p_r[1]
  tx1 = p[:, :64]
  agg = p[:, 64:]
  hp0 = g0_r[:, :64]
  xp0 = xp0_r[...]
  o1 = _dotT(hp0, cl0_r[...]) + _dotT(tx1, cl1_r[...]) + cb_r[...]
  o1 = jnp.where(o1 >= 0, o1, 0.01 * o1)
  o2 = agg + _dot(xp0, arW_r[...]) + ab_r[...]
  o2 = jnp.maximum(o2, 0.0)
  o3 = _dotT(o1 + o2, lW_r[...]) + lb_r[...]
  xp1 = _dotT(o3, pxW_r[...]) + pxb_r[...]
  hp1 = _dotT(x_r[...], phW_r[...]) + phb_r[...]
  hh1 = _dot(xp1, aiW_r[...])
  g1_r[...] = jnp.concatenate([hp1, hh1], axis=1)
  xp1_r[...] = xp1


def _tc3_body(g1_r, xp1_r, p_r,
              cl0_r, cl1_r, cb_r, arW_r, ab_r, lW_r, lb_r,
              clsW_r, clsb_r, out_r):
  p = p_r[0] + p_r[1]
  tx1 = p[:, :64]
  agg = p[:, 64:]
  hp1 = g1_r[:, :64]
  o1 = _dotT(hp1, cl0_r[...]) + _dotT(tx1, cl1_r[...]) + cb_r[...]
  o1 = jnp.where(o1 >= 0, o1, 0.01 * o1)
  o2 = agg + _dot(xp1_r[...], arW_r[...]) + ab_r[...]
  o2 = jnp.maximum(o2, 0.0)
  o3 = _dotT(o1 + o2, lW_r[...]) + lb_r[...]
  logits = _dotT(o3, clsW_r[...]) + clsb_r[...]
  m = jnp.max(logits, axis=1, keepdims=True)
  sh = logits - m
  out_r[...] = sh - jnp.log(jnp.sum(jnp.exp(sh), axis=1, keepdims=True))


def _full(shape):
  return pl.BlockSpec(shape, lambda i: (0,) * len(shape))


def _rows(shape):
  return pl.BlockSpec(shape, lambda i: (i,) + (0,) * (len(shape) - 1))


def kernel(x, edge_index, edge_weight,
           c0_pre_h_W, c0_pre_h_b, c0_pre_x_W, c0_pre_x_b,
           c0_cheb_lin0_W, c0_cheb_lin1_W, c0_cheb_b,
           c0_arma_init_W, c0_arma_root_W, c0_arma_b,
           c0_lin_W, c0_lin_b,
           c1_pre_h_W, c1_pre_h_b, c1_pre_x_W, c1_pre_x_b,
           c1_cheb_lin0_W, c1_cheb_lin1_W, c1_cheb_b,
           c1_arma_init_W, c1_arma_root_W, c1_arma_b,
           c1_lin_W, c1_lin_b,
           cls_W, cls_b):
  r2 = lambda b: b.reshape(1, -1)

  pad_idx = (jnp.arange(PAD, dtype=jnp.int32) * 997) % N
  src = jnp.concatenate([edge_index[0], pad_idx])
  dst = jnp.concatenate([edge_index[1], pad_idx])
  ew = jnp.concatenate([edge_weight, jnp.zeros((PAD,), jnp.float32)])

  g0, xp0 = pl.pallas_call(
      _tc1_body,
      grid=(GRID,),
      in_specs=[_rows((RB, 128)), _full((64, 128)), _full((1, 64)),
                _full((64, 128)), _full((1, 64)), _full((64, 64))],
      out_specs=[_rows((RB, 128)), _rows((RB, 64))],
      out_shape=[jax.ShapeDtypeStruct((N, 128), jnp.float32),
                 jax.ShapeDtypeStruct((N, 64), jnp.float32)],
  )(x, c0_pre_x_W, r2(c0_pre_x_b), c0_pre_h_W, r2(c0_pre_h_b),
    c0_arma_init_W)

  p0, normc, norma = _sc_pass1(src, dst, ew, g0)

  g1, xp1 = pl.pallas_call(
      _tc2_body,
      grid=(GRID,),
      in_specs=[_rows((RB, 128)), _rows((RB, 128)), _rows((RB, 64)),
                pl.BlockSpec((2, RB, 128), lambda i: (0, i, 0)),
                _full((64, 64)), _full((64, 64)), _full((1, 64)),
                _full((64, 64)), _full((1, 64)),
                _full((64, 64)), _full((1, 64)),
                _full((64, 64)), _full((1, 64)),
                _full((64, 128)), _full((1, 64)), _full((64, 64))],
      out_specs=[_rows((RB, 128)), _rows((RB, 64))],
      out_shape=[jax.ShapeDtypeStruct((N, 128), jnp.float32),
                 jax.ShapeDtypeStruct((N, 64), jnp.float32)],
  )(x, g0, xp0, p0,
    c0_cheb_lin0_W, c0_cheb_lin1_W, r2(c0_cheb_b),
    c0_arma_root_W, r2(c0_arma_b), c0_lin_W, r2(c0_lin_b),
    c1_pre_x_W, r2(c1_pre_x_b), c1_pre_h_W, r2(c1_pre_h_b),
    c1_arma_init_W)

  p1 = _sc_pass2(src, dst, normc, norma, g1)

  out = pl.pallas_call(
      _tc3_body,
      grid=(GRID,),
      in_specs=[_rows((RB, 128)), _rows((RB, 64)),
                pl.BlockSpec((2, RB, 128), lambda i: (0, i, 0)),
                _full((64, 64)), _full((64, 64)), _full((1, 64)),
                _full((64, 64)), _full((1, 64)),
                _full((64, 64)), _full((1, 64)),
                _full((32, 64)), _full((1, 32))],
      out_specs=_rows((RB, NC_CLS)),
      out_shape=jax.ShapeDtypeStruct((N, NC_CLS), jnp.float32),
  )(g1, xp1, p1,
    c1_cheb_lin0_W, c1_cheb_lin1_W, r2(c1_cheb_b),
    c1_arma_root_W, r2(c1_arma_b), c1_lin_W, r2(c1_lin_b),
    cls_W, r2(cls_b))

  return out


# speedup vs baseline: 28.2134x; 1.1139x over previous
import jax
import jax.numpy as jnp
from jax import lax
from jax.experimental import pallas as pl
from jax.experimental.pallas import tpu as pltpu
from jax.experimental.pallas import tpu_sc as plsc

N = 10000
NC_CLS = 32
E = 320000

NCORES = 2
NSUB = 16
NW = NCORES * NSUB

SUB = 128
CH_P = 128
CH_D = 512
NSC_D = CH_D // SUB

EP = 327680
PAD = EP - E
E_DEG = EP // NSUB
E_PASS = EP // NW
DEG_CHUNKS = E_DEG // CH_D
PASS_PAIRS = E_PASS // (2 * CH_P)

DN = 10240
DROWS = DN // NSUB


def _newton_rsqrt(v):
  b = lax.bitcast_convert_type(v, jnp.int32)
  h = jnp.int32(0x5F3759DF) - (b >> 1)
  y = lax.bitcast_convert_type(h, jnp.float32)
  for _ in range(4):
    y = y * (1.5 - 0.5 * v * y * y)
  return y


def _zero_vmem2d(ref, rows, cols):
  z = jnp.zeros((16,), jnp.float32)
  for r in range(rows):
    for c0 in range(cols // 16):
      ref[r, pl.ds(c0 * 16, 16)] = z


def _zero_acc(acc_sh, zrow, s, sem):
  _zero_vmem2d(zrow, 16, 128)
  rbase = s * DROWS
  descs = []
  for j in range(DROWS // 16):
    descs.append(pltpu.async_copy(
        zrow, acc_sh.at[pl.ds(rbase + j * 16, 16), :], sem))
  _drain(descs)


def _drain(descs):
  for d in descs:
    d.wait()


def _scale_rows(rows_ref, nc_ref, na_ref):
  @pl.loop(0, CH_P // 16)
  def _(g):
    base = pl.multiple_of(g * 16, 16)
    ncv = nc_ref[pl.ds(base, 16)]
    nav = na_ref[pl.ds(base, 16)]
    for i in range(16):
      e = base + i
      ncs = ncv[i]
      nas = nav[i]
      for f in range(4):
        rows_ref[e, pl.ds(f * 16, 16)] = rows_ref[e, pl.ds(f * 16, 16)] * ncs
      for f in range(4, 8):
        rows_ref[e, pl.ds(f * 16, 16)] = rows_ref[e, pl.ds(f * 16, 16)] * nas


def _make_sc_pass1():
  mesh = plsc.VectorSubcoreMesh(
      core_axis_name="c", subcore_axis_name="s",
      num_cores=NCORES, num_subcores=NSUB)

  def body(src_hbm, dst_hbm, ew_hbm, g_hbm, p_hbm, normc_hbm, norma_hbm,
           acc_sh, degc_sh, dega_sh,
           zrow, dwork, wdeg, wcdeg,
           ds0, ds1, ds2, ds3, dd0, dd1, dd2, dd3,
           ps0, ps1, pd0, pd1, wp0, wp1, nc0, nc1, na0, na1,
           dcs0, dcs1, dcd0, dcd1, das0, das1, dad0, dad1,
           rows0, rows1,
           sem_z, sem_dl, sem_dsc,
           sem_in0, sem_in1, sem_g0, sem_g1, sem_r0, sem_r1,
           sem_o0, sem_o1, sem_n0, sem_n1):
    c = lax.axis_index("c")
    s = lax.axis_index("s")
    dsrc = [ds0, ds1, ds2, ds3]
    ddst = [dd0, dd1, dd2, dd3]

    _zero_acc(acc_sh, zrow, s, sem_z)
    zflat = zrow.at[0]
    descs = []
    for j in range(DROWS // 128):
      descs.append(pltpu.async_copy(
          zflat, degc_sh.at[pl.ds(s * DROWS + j * 128, 128)], sem_z))
      descs.append(pltpu.async_copy(
          zflat, dega_sh.at[pl.ds(s * DROWS + j * 128, 128)], sem_z))
    _drain(descs)
    plsc.subcore_barrier()

    dbase = s * E_DEG

    @pl.loop(0, DEG_CHUNKS)
    def _(j):
      off = pl.multiple_of(dbase + j * CH_D, CH_D)
      descs = []
      for k in range(NSC_D):
        descs.append(pltpu.async_copy(
            src_hbm.at[pl.ds(off + k * SUB, SUB)], dsrc[k], sem_dl))
        descs.append(pltpu.async_copy(
            dst_hbm.at[pl.ds(off + k * SUB, SUB)], ddst[k], sem_dl))
      descs.append(pltpu.async_copy(ew_hbm.at[pl.ds(off, CH_D)], wdeg, sem_dl))
      _drain(descs)
      for g in range(CH_D // 16):
        sl = pl.ds(g * 16, 16)
        k, col = divmod(g * 16, SUB)
        sv = dsrc[k][pl.ds(col, 16)]
        dv = ddst[k][pl.ds(col, 16)]
        wcdeg[sl] = jnp.where(sv == dv, 0.0, wdeg[sl])
      descs = []
      for k in range(NSC_D):
        descs.append(pltpu.async_copy(
            wcdeg.at[pl.ds(k * SUB, SUB)], degc_sh.at[dsrc[k]],
            sem_dsc, add=True))
        descs.append(pltpu.async_copy(
            wdeg.at[pl.ds(k * SUB, SUB)], dega_sh.at[ddst[k]],
            sem_dsc, add=True))
      _drain(descs)

    plsc.subcore_barrier()

    rbase = s * DROWS
    for deg_sh in (degc_sh, dega_sh):
      pltpu.sync_copy(deg_sh.at[pl.ds(rbase, DROWS)], dwork)
      @pl.loop(0, DROWS // 16)
      def _(i):
        sl = pl.ds(pl.multiple_of(i * 16, 16), 16)
        v = dwork[sl]
        dwork[sl] = jnp.where(v > 0, _newton_rsqrt(v), 0.0)
      pltpu.sync_copy(dwork, deg_sh.at[pl.ds(rbase, DROWS)])
    plsc.subcore_barrier()

    ebase = (c * NSUB + s) * E_PASS
    sets = [
        dict(ps=ps0, pd=pd0, wp=wp0, nc=nc0, na=na0, dcs=dcs0, dcd=dcd0,
             das=das0, dad=dad0, rows=rows0, sem_in=sem_in0, sem_g=sem_g0,
             sem_r=sem_r0, sem_o=sem_o0, sem_n=sem_n0),
        dict(ps=ps1, pd=pd1, wp=wp1, nc=nc1, na=na1, dcs=dcs1, dcd=dcd1,
             das=das1, dad=dad1, rows=rows1, sem_in=sem_in1, sem_g=sem_g1,
             sem_r=sem_r1, sem_o=sem_o1, sem_n=sem_n1),
    ]

    def fire_loads(off, S):
      return [
          pltpu.async_copy(src_hbm.at[pl.ds(off, CH_P)], S['ps'], S['sem_in']),
          pltpu.async_copy(dst_hbm.at[pl.ds(off, CH_P)], S['pd'], S['sem_in']),
          pltpu.async_copy(ew_hbm.at[pl.ds(off, CH_P)], S['wp'], S['sem_in']),
      ]

    def fire_gathers(S):
      g = [
          pltpu.async_copy(degc_sh.at[S['ps']], S['dcs'], S['sem_g']),
          pltpu.async_copy(degc_sh.at[S['pd']], S['dcd'], S['sem_g']),
          pltpu.async_copy(dega_sh.at[S['ps']], S['das'], S['sem_g']),
          pltpu.async_copy(dega_sh.at[S['pd']], S['dad'], S['sem_g']),
      ]
      r = [pltpu.async_copy(g_hbm.at[S['ps']], S['rows'], S['sem_r'])]
      return g, r

    def compute_and_out(off, S):
      for g in range(CH_P // 16):
        sl = pl.ds(g * 16, 16)
        sv = S['ps'][sl]
        dv = S['pd'][sl]
        wv = S['wp'][sl]
        wc = jnp.where(sv == dv, 0.0, wv)
        S['nc'][sl] = -(S['dcs'][sl] * wc * S['dcd'][sl])
        S['na'][sl] = S['das'][sl] * wv * S['dad'][sl]
      nw = [
          pltpu.async_copy(S['nc'], normc_hbm.at[pl.ds(off, CH_P)], S['sem_n']),
          pltpu.async_copy(S['na'], norma_hbm.at[pl.ds(off, CH_P)], S['sem_n']),
      ]
      return nw

    def fire_scatter(S):
      return [pltpu.async_copy(S['rows'], acc_sh.at[S['pd']],
                               S['sem_o'], add=True)]

    @pl.loop(0, PASS_PAIRS)
    def _(t):
      off0 = pl.multiple_of(ebase + t * (2 * CH_P), CH_P)
      off1 = pl.multiple_of(ebase + t * (2 * CH_P) + CH_P, CH_P)
      S0, S1 = sets
      l0 = fire_loads(off0, S0)
      l1 = fire_loads(off1, S1)
      _drain(l0)
      g0, r0 = fire_gathers(S0)
      _drain(l1)
      g1, r1 = fire_gathers(S1)
      _drain(g0)
      nw0 = compute_and_out(off0, S0)
      _drain(r0)
      _scale_rows(S0['rows'], S0['nc'], S0['na'])
      s0 = fire_scatter(S0)
      _drain(g1)
      nw1 = compute_and_out(off1, S1)
      _drain(r1)
      _scale_rows(S1['rows'], S1['nc'], S1['na'])
      s1 = fire_scatter(S1)
      _drain(nw0)
      _drain(s0)
      _drain(nw1)
      _drain(s1)

    plsc.subcore_barrier()

    @pl.when(s < NSUB - 1)
    def _():
      pltpu.sync_copy(acc_sh.at[pl.ds(rbase, DROWS), :],
                      p_hbm.at[c, pl.ds(rbase, DROWS), :])
    @pl.when(s == NSUB - 1)
    def _():
      pltpu.sync_copy(acc_sh.at[pl.ds(rbase, N - (NSUB - 1) * DROWS), :],
                      p_hbm.at[c, pl.ds(rbase, N - (NSUB - 1) * DROWS), :])

  sems = [pltpu.SemaphoreType.DMA] * 13
  return pl.kernel(
      body,
      out_type=(
          jax.ShapeDtypeStruct((NCORES, N, 128), jnp.float32),
          jax.ShapeDtypeStruct((EP,), jnp.float32),
          jax.ShapeDtypeStruct((EP,), jnp.float32),
      ),
      mesh=mesh,
      compiler_params=pltpu.CompilerParams(use_tc_tiling_on_sc=False),
      scratch_types=[
          pltpu.VMEM_SHARED((DN, 128), jnp.float32),
          pltpu.VMEM_SHARED((DN,), jnp.float32),
          pltpu.VMEM_SHARED((DN,), jnp.float32),
          pltpu.VMEM((16, 128), jnp.float32),
          pltpu.VMEM((DROWS,), jnp.float32),
          pltpu.VMEM((CH_D,), jnp.float32),
          pltpu.VMEM((CH_D,), jnp.float32),
      ] + [pltpu.VMEM((SUB,), jnp.int32)] * 8
        + [pltpu.VMEM((CH_P,), jnp.int32)] * 4
        + [pltpu.VMEM((CH_P,), jnp.float32)] * 14
        + [pltpu.VMEM((CH_P, 128), jnp.float32)] * 2
        + sems,
      name="sc_deg_norm_pass0",
  )


def _make_sc_pass2():
  mesh = plsc.VectorSubcoreMesh(
      core_axis_name="c", subcore_axis_name="s",
      num_cores=NCORES, num_subcores=NSUB)

  def body(src_hbm, dst_hbm, normc_hbm, norma_hbm, g_hbm, p_hbm,
           acc_sh, zrow,
           ps0, ps1, pd0, pd1, nc0, nc1, na0, na1,
           rows0, rows1,
           sem_z, sem_in0, sem_in1, sem_r0, sem_r1, sem_o0, sem_o1):
    c = lax.axis_index("c")
    s = lax.axis_index("s")

    _zero_acc(acc_sh, zrow, s, sem_z)
    plsc.subcore_barrier()

    ebase = (c * NSUB + s) * E_PASS
    sets = [
        dict(ps=ps0, pd=pd0, nc=nc0, na=na0, rows=rows0,
             sem_in=sem_in0, sem_r=sem_r0, sem_o=sem_o0),
        dict(ps=ps1, pd=pd1, nc=nc1, na=na1, rows=rows1,
             sem_in=sem_in1, sem_r=sem_r1, sem_o=sem_o1),
    ]

    def fire_loads(off, S):
      return [
          pltpu.async_copy(src_hbm.at[pl.ds(off, CH_P)], S['ps'], S['sem_in']),
          pltpu.async_copy(dst_hbm.at[pl.ds(off, CH_P)], S['pd'], S['sem_in']),
          pltpu.async_copy(normc_hbm.at[pl.ds(off, CH_P)], S['nc'],
                           S['sem_in']),
          pltpu.async_copy(norma_hbm.at[pl.ds(off, CH_P)], S['na'],
                           S['sem_in']),
      ]

    @pl.loop(0, PASS_PAIRS)
    def _(t):
      off0 = pl.multiple_of(ebase + t * (2 * CH_P), CH_P)
      off1 = pl.multiple_of(ebase + t * (2 * CH_P) + CH_P, CH_P)
      S0, S1 = sets
      l0 = fire_loads(off0, S0)
      l1 = fire_loads(off1, S1)
      _drain(l0)
      r0 = [pltpu.async_copy(g_hbm.at[S0['ps']], S0['rows'], S0['sem_r'])]
      _drain(l1)
      r1 = [pltpu.async_copy(g_hbm.at[S1['ps']], S1['rows'], S1['sem_r'])]
      _drain(r0)
      _scale_rows(S0['rows'], S0['nc'], S0['na'])
      s0 = [pltpu.async_copy(S0['rows'], acc_sh.at[S0['pd']],
                             S0['sem_o'], add=True)]
      _drain(r1)
      _scale_rows(S1['rows'], S1['nc'], S1['na'])
      s1 = [pltpu.async_copy(S1['rows'], acc_sh.at[S1['pd']],
                             S1['sem_o'], add=True)]
      _drain(s0)
      _drain(s1)

    plsc.subcore_barrier()
    rbase = s * DROWS
    @pl.when(s < NSUB - 1)
    def _():
      pltpu.sync_copy(acc_sh.at[pl.ds(rbase, DROWS), :],
                      p_hbm.at[c, pl.ds(rbase, DROWS), :])
    @pl.when(s == NSUB - 1)
    def _():
      pltpu.sync_copy(acc_sh.at[pl.ds(rbase, N - (NSUB - 1) * DROWS), :],
                      p_hbm.at[c, pl.ds(rbase, N - (NSUB - 1) * DROWS), :])

  return pl.kernel(
      body,
      out_type=jax.ShapeDtypeStruct((NCORES, N, 128), jnp.float32),
      mesh=mesh,
      compiler_params=pltpu.CompilerParams(use_tc_tiling_on_sc=False),
      scratch_types=[
          pltpu.VMEM_SHARED((DN, 128), jnp.float32),
          pltpu.VMEM((16, 128), jnp.float32),
      ] + [pltpu.VMEM((CH_P,), jnp.int32)] * 4
        + [pltpu.VMEM((CH_P,), jnp.float32)] * 4
        + [pltpu.VMEM((CH_P, 128), jnp.float32)] * 2
        + [pltpu.SemaphoreType.DMA] * 7,
      name="sc_pass1",
  )


_sc_pass1 = _make_sc_pass1()
_sc_pass2 = _make_sc_pass2()


RB = 1000
GRID = N // RB


def _dotT(a, w):
  return lax.dot_general(a, w, (((1,), (1,)), ((), ())),
                         preferred_element_type=jnp.float32)


def _dot(a, w):
  return lax.dot_general(a, w, (((1,), (0,)), ((), ())),
                         preferred_element_type=jnp.float32)


def _tc1_body(x_r, pxW_r, pxb_r, phW_r, phb_r, aiW_r, g_r, xp_r):
  xb = x_r[...]
  xp = _dotT(xb, pxW_r[...]) + pxb_r[...]
  hp = _dotT(xb, phW_r[...]) + phb_r[...]
  hh = _dot(xp, aiW_r[...])
  g_r[...] = jnp.concatenate([hp, hh], axis=1)
  xp_r[...] = xp


def _tc2_body(x_r, g0_r, xp0_r, p_r,
              cl0_r, cl1_r, cb_r, arW_r, ab_r, lW_r, lb_r,
              pxW_r, pxb_r, phW_r, phb_r, aiW_r,
              g1_r, xp1_r):
  p = p_r[0] + p_r[1]
  tx1 = p[:, :64]
  agg = p[:, 64:]
  hp0 = g0_r[:, :64]
  xp0 = xp0_r[...]
  o1 = _dotT(hp0, cl0_r[...]) + _dotT(tx1, cl1_r[...]) + cb_r[...]
  o1 = jnp.where(o1 >= 0, o1, 0.01 * o1)
  o2 = agg + _dot(xp0, arW_r[...]) + ab_r[...]
  o2 = jnp.maximum(o2, 0.0)
  o3 = _dotT(o1 + o2, lW_r[...]) + lb_r[...]
  xp1 = _dotT(o3, pxW_r[...]) + pxb_r[...]
  hp1 = _dotT(x_r[...], phW_r[...]) + phb_r[...]
  hh1 = _dot(xp1, aiW_r[...])
  g1_r[...] = jnp.concatenate([hp1, hh1], axis=1)
  xp1_r[...] = xp1


def _tc3_body(g1_r, xp1_r, p_r,
              cl0_r, cl1_r, cb_r, arW_r, ab_r, lW_r, lb_r,
              clsW_r, clsb_r, out_r):
  p = p_r[0] + p_r[1]
  tx1 = p[:, :64]
  agg = p[:, 64:]
  hp1 = g1_r[:, :64]
  o1 = _dotT(hp1, cl0_r[...]) + _dotT(tx1, cl1_r[...]) + cb_r[...]
  o1 = jnp.where(o1 >= 0, o1, 0.01 * o1)
  o2 = agg + _dot(xp1_r[...], arW_r[...]) + ab_r[...]
  o2 = jnp.maximum(o2, 0.0)
  o3 = _dotT(o1 + o2, lW_r[...]) + lb_r[...]
  logits = _dotT(o3, clsW_r[...]) + clsb_r[...]
  m = jnp.max(logits, axis=1, keepdims=True)
  sh = logits - m
  out_r[...] = sh - jnp.log(jnp.sum(jnp.exp(sh), axis=1, keepdims=True))


def _full(shape):
  return pl.BlockSpec(shape, lambda i: (0,) * len(shape))


def _rows(shape):
  return pl.BlockSpec(shape, lambda i: (i,) + (0,) * (len(shape) - 1))


def kernel(x, edge_index, edge_weight,
           c0_pre_h_W, c0_pre_h_b, c0_pre_x_W, c0_pre_x_b,
           c0_cheb_lin0_W, c0_cheb_lin1_W, c0_cheb_b,
           c0_arma_init_W, c0_arma_root_W, c0_arma_b,
           c0_lin_W, c0_lin_b,
           c1_pre_h_W, c1_pre_h_b, c1_pre_x_W, c1_pre_x_b,
           c1_cheb_lin0_W, c1_cheb_lin1_W, c1_cheb_b,
           c1_arma_init_W, c1_arma_root_W, c1_arma_b,
           c1_lin_W, c1_lin_b,
           cls_W, cls_b):
  r2 = lambda b: b.reshape(1, -1)

  pad_idx = (jnp.arange(PAD, dtype=jnp.int32) * 997) % N
  src = jnp.concatenate([edge_index[0], pad_idx])
  dst = jnp.concatenate([edge_index[1], pad_idx])
  ew = jnp.concatenate([edge_weight, jnp.zeros((PAD,), jnp.float32)])

  g0, xp0 = pl.pallas_call(
      _tc1_body,
      grid=(GRID,),
      in_specs=[_rows((RB, 128)), _full((64, 128)), _full((1, 64)),
                _full((64, 128)), _full((1, 64)), _full((64, 64))],
      out_specs=[_rows((RB, 128)), _rows((RB, 64))],
      out_shape=[jax.ShapeDtypeStruct((N, 128), jnp.float32),
                 jax.ShapeDtypeStruct((N, 64), jnp.float32)],
  )(x, c0_pre_x_W, r2(c0_pre_x_b), c0_pre_h_W, r2(c0_pre_h_b),
    c0_arma_init_W)

  p0, normc, norma = _sc_pass1(src, dst, ew, g0)

  g1, xp1 = pl.pallas_call(
      _tc2_body,
      grid=(GRID,),
      in_specs=[_rows((RB, 128)), _rows((RB, 128)), _rows((RB, 64)),
                pl.BlockSpec((2, RB, 128), lambda i: (0, i, 0)),
                _full((64, 64)), _full((64, 64)), _full((1, 64)),
                _full((64, 64)), _full((1, 64)),
                _full((64, 64)), _full((1, 64)),
                _full((64, 64)), _full((1, 64)),
                _full((64, 128)), _full((1, 64)), _full((64, 64))],
      out_specs=[_rows((RB, 128)), _rows((RB, 64))],
      out_shape=[jax.ShapeDtypeStruct((N, 128), jnp.float32),
                 jax.ShapeDtypeStruct((N, 64), jnp.float32)],
  )(x, g0, xp0, p0,
    c0_cheb_lin0_W, c0_cheb_lin1_W, r2(c0_cheb_b),
    c0_arma_root_W, r2(c0_arma_b), c0_lin_W, r2(c0_lin_b),
    c1_pre_x_W, r2(c1_pre_x_b), c1_pre_h_W, r2(c1_pre_h_b),
    c1_arma_init_W)

  p1 = _sc_pass2(src, dst, normc, norma, g1)

  out = pl.pallas_call(
      _tc3_body,
      grid=(GRID,),
      in_specs=[_rows((RB, 128)), _rows((RB, 64)),
                pl.BlockSpec((2, RB, 128), lambda i: (0, i, 0)),
                _full((64, 64)), _full((64, 64)), _full((1, 64)),
                _full((64, 64)), _full((1, 64)),
                _full((64, 64)), _full((1, 64)),
                _full((32, 64)), _full((1, 32))],
      out_specs=_rows((RB, NC_CLS)),
      out_shape=jax.ShapeDtypeStruct((N, NC_CLS), jnp.float32),
  )(g1, xp1, p1,
    c1_cheb_lin0_W, c1_cheb_lin1_W, r2(c1_cheb_b),
    c1_arma_root_W, r2(c1_arma_b), c1_lin_W, r2(c1_lin_b),
    cls_W, r2(cls_b))

  return out


# speedup vs baseline: 28.4495x; 1.0084x over previous
import jax
import jax.numpy as jnp
from jax import lax
from jax.experimental import pallas as pl
from jax.experimental.pallas import tpu as pltpu
from jax.experimental.pallas import tpu_sc as plsc

N = 10000
NC_CLS = 32
E = 320000

NCORES = 2
NSUB = 16
NW = NCORES * NSUB

SUB = 128
CH_P = 128
CH_D = 512
NSC_D = CH_D // SUB

EP = 327680
PAD = EP - E
E_DEG = EP // NSUB
E_PASS = EP // NW
DEG_CHUNKS = E_DEG // CH_D
PASS_PAIRS = E_PASS // (2 * CH_P)

DN = 10240
DROWS = DN // NSUB


def _newton_rsqrt(v):
  b = lax.bitcast_convert_type(v, jnp.int32)
  h = jnp.int32(0x5F3759DF) - (b >> 1)
  y = lax.bitcast_convert_type(h, jnp.float32)
  for _ in range(4):
    y = y * (1.5 - 0.5 * v * y * y)
  return y


def _zero_vmem2d(ref, rows, cols):
  z = jnp.zeros((16,), jnp.float32)
  for r in range(rows):
    for c0 in range(cols // 16):
      ref[r, pl.ds(c0 * 16, 16)] = z


def _zero_acc(acc_sh, zrow, s, sem):
  _zero_vmem2d(zrow, 16, 128)
  rbase = s * DROWS
  descs = []
  for j in range(DROWS // 16):
    descs.append(pltpu.async_copy(
        zrow, acc_sh.at[pl.ds(rbase + j * 16, 16), :], sem))
  _drain(descs)


def _drain(descs):
  for d in descs:
    d.wait()


def _scale_rows(rows_ref, nc_ref, na_ref):
  @pl.loop(0, CH_P // 16)
  def _(g):
    base = pl.multiple_of(g * 16, 16)
    ncv = nc_ref[pl.ds(base, 16)]
    nav = na_ref[pl.ds(base, 16)]
    for i in range(16):
      e = base + i
      ncs = ncv[i]
      nas = nav[i]
      for f in range(4):
        rows_ref[e, pl.ds(f * 16, 16)] = rows_ref[e, pl.ds(f * 16, 16)] * ncs
      for f in range(4, 8):
        rows_ref[e, pl.ds(f * 16, 16)] = rows_ref[e, pl.ds(f * 16, 16)] * nas


def _make_sc_pass1():
  mesh = plsc.VectorSubcoreMesh(
      core_axis_name="c", subcore_axis_name="s",
      num_cores=NCORES, num_subcores=NSUB)

  def body(src_hbm, dst_hbm, ew_hbm, g_hbm, p_hbm, normc_hbm, norma_hbm,
           acc_sh, degc_sh, dega_sh,
           zrow, dwork, wdeg, wcdeg,
           ds0, ds1, ds2, ds3, dd0, dd1, dd2, dd3,
           ps0, ps1, pd0, pd1, wp0, wp1, nc0, nc1, na0, na1,
           dcs0, dcs1, dcd0, dcd1, das0, das1, dad0, dad1,
           rows0, rows1,
           sem_z, sem_dl, sem_dsc,
           sem_in0, sem_in1, sem_g0, sem_g1, sem_r0, sem_r1,
           sem_o0, sem_o1, sem_n0, sem_n1):
    c = lax.axis_index("c")
    s = lax.axis_index("s")
    dsrc = [ds0, ds1, ds2, ds3]
    ddst = [dd0, dd1, dd2, dd3]

    _zero_acc(acc_sh, zrow, s, sem_z)
    zflat = zrow.at[0]
    descs = []
    for j in range(DROWS // 128):
      descs.append(pltpu.async_copy(
          zflat, degc_sh.at[pl.ds(s * DROWS + j * 128, 128)], sem_z))
      descs.append(pltpu.async_copy(
          zflat, dega_sh.at[pl.ds(s * DROWS + j * 128, 128)], sem_z))
    _drain(descs)
    plsc.subcore_barrier()

    dbase = s * E_DEG
    dsets = [
        dict(sr=[ds0, ds1], dr=[dd0, dd1], woff=0, sem_l=sem_dl,
             sem_s=sem_dsc),
        dict(sr=[ds2, ds3], dr=[dd2, dd3], woff=CH_D // 2, sem_l=sem_in0,
             sem_s=sem_o0),
    ]

    def deg_loads(off, S):
      descs = []
      for k in range(2):
        descs.append(pltpu.async_copy(
            src_hbm.at[pl.ds(off + k * SUB, SUB)], S['sr'][k], S['sem_l']))
        descs.append(pltpu.async_copy(
            dst_hbm.at[pl.ds(off + k * SUB, SUB)], S['dr'][k], S['sem_l']))
      descs.append(pltpu.async_copy(
          ew_hbm.at[pl.ds(off, CH_D // 2)],
          wdeg.at[pl.ds(S['woff'], CH_D // 2)], S['sem_l']))
      return descs

    def deg_compute_scatter(S):
      for g in range(CH_D // 2 // 16):
        k, col = divmod(g * 16, SUB)
        sl = pl.ds(S['woff'] + g * 16, 16)
        sv = S['sr'][k][pl.ds(col, 16)]
        dv = S['dr'][k][pl.ds(col, 16)]
        wcdeg[sl] = jnp.where(sv == dv, 0.0, wdeg[sl])
      descs = []
      for k in range(2):
        descs.append(pltpu.async_copy(
            wcdeg.at[pl.ds(S['woff'] + k * SUB, SUB)], degc_sh.at[S['sr'][k]],
            S['sem_s'], add=True))
        descs.append(pltpu.async_copy(
            wdeg.at[pl.ds(S['woff'] + k * SUB, SUB)], dega_sh.at[S['dr'][k]],
            S['sem_s'], add=True))
      return descs

    @pl.loop(0, DEG_CHUNKS)
    def _(j):
      off = pl.multiple_of(dbase + j * CH_D, CH_D)
      A, B = dsets
      lA = deg_loads(off, A)
      lB = deg_loads(off + CH_D // 2, B)
      _drain(lA)
      sA = deg_compute_scatter(A)
      _drain(lB)
      sB = deg_compute_scatter(B)
      _drain(sA)
      _drain(sB)

    plsc.subcore_barrier()

    rbase = s * DROWS
    for deg_sh in (degc_sh, dega_sh):
      pltpu.sync_copy(deg_sh.at[pl.ds(rbase, DROWS)], dwork)
      @pl.loop(0, DROWS // 16)
      def _(i):
        sl = pl.ds(pl.multiple_of(i * 16, 16), 16)
        v = dwork[sl]
        dwork[sl] = jnp.where(v > 0, _newton_rsqrt(v), 0.0)
      pltpu.sync_copy(dwork, deg_sh.at[pl.ds(rbase, DROWS)])
    plsc.subcore_barrier()

    ebase = (c * NSUB + s) * E_PASS
    sets = [
        dict(ps=ps0, pd=pd0, wp=wp0, nc=nc0, na=na0, dcs=dcs0, dcd=dcd0,
             das=das0, dad=dad0, rows=rows0, sem_in=sem_in0, sem_g=sem_g0,
             sem_r=sem_r0, sem_o=sem_o0, sem_n=sem_n0),
        dict(ps=ps1, pd=pd1, wp=wp1, nc=nc1, na=na1, dcs=dcs1, dcd=dcd1,
             das=das1, dad=dad1, rows=rows1, sem_in=sem_in1, sem_g=sem_g1,
             sem_r=sem_r1, sem_o=sem_o1, sem_n=sem_n1),
    ]

    def fire_loads(off, S):
      return [
          pltpu.async_copy(src_hbm.at[pl.ds(off, CH_P)], S['ps'], S['sem_in']),
          pltpu.async_copy(dst_hbm.at[pl.ds(off, CH_P)], S['pd'], S['sem_in']),
          pltpu.async_copy(ew_hbm.at[pl.ds(off, CH_P)], S['wp'], S['sem_in']),
      ]

    def fire_gathers(S):
      g = [
          pltpu.async_copy(degc_sh.at[S['ps']], S['dcs'], S['sem_g']),
          pltpu.async_copy(degc_sh.at[S['pd']], S['dcd'], S['sem_g']),
          pltpu.async_copy(dega_sh.at[S['ps']], S['das'], S['sem_g']),
          pltpu.async_copy(dega_sh.at[S['pd']], S['dad'], S['sem_g']),
      ]
      r = [pltpu.async_copy(g_hbm.at[S['ps']], S['rows'], S['sem_r'])]
      return g, r

    def compute_and_out(off, S):
      for g in range(CH_P // 16):
        sl = pl.ds(g * 16, 16)
        sv = S['ps'][sl]
        dv = S['pd'][sl]
        wv = S['wp'][sl]
        wc = jnp.where(sv == dv, 0.0, wv)
        S['nc'][sl] = -(S['dcs'][sl] * wc * S['dcd'][sl])
        S['na'][sl] = S['das'][sl] * wv * S['dad'][sl]
      nw = [
          pltpu.async_copy(S['nc'], normc_hbm.at[pl.ds(off, CH_P)], S['sem_n']),
          pltpu.async_copy(S['na'], norma_hbm.at[pl.ds(off, CH_P)], S['sem_n']),
      ]
      return nw

    def fire_scatter(S):
      return [pltpu.async_copy(S['rows'], acc_sh.at[S['pd']],
                               S['sem_o'], add=True)]

    @pl.loop(0, PASS_PAIRS)
    def _(t):
      off0 = pl.multiple_of(ebase + t * (2 * CH_P), CH_P)
      off1 = pl.multiple_of(ebase + t * (2 * CH_P) + CH_P, CH_P)
      S0, S1 = sets
      l0 = fire_loads(off0, S0)
      l1 = fire_loads(off1, S1)
      _drain(l0)
      g0, r0 = fire_gathers(S0)
      _drain(l1)
      g1, r1 = fire_gathers(S1)
      _drain(g0)
      nw0 = compute_and_out(off0, S0)
      _drain(r0)
      _scale_rows(S0['rows'], S0['nc'], S0['na'])
      s0 = fire_scatter(S0)
      _drain(g1)
      nw1 = compute_and_out(off1, S1)
      _drain(r1)
      _scale_rows(S1['rows'], S1['nc'], S1['na'])
      s1 = fire_scatter(S1)
      _drain(nw0)
      _drain(s0)
      _drain(nw1)
      _drain(s1)

    plsc.subcore_barrier()

    @pl.when(s < NSUB - 1)
    def _():
      pltpu.sync_copy(acc_sh.at[pl.ds(rbase, DROWS), :],
                      p_hbm.at[c, pl.ds(rbase, DROWS), :])
    @pl.when(s == NSUB - 1)
    def _():
      pltpu.sync_copy(acc_sh.at[pl.ds(rbase, N - (NSUB - 1) * DROWS), :],
                      p_hbm.at[c, pl.ds(rbase, N - (NSUB - 1) * DROWS), :])

  sems = [pltpu.SemaphoreType.DMA] * 13
  return pl.kernel(
      body,
      out_type=(
          jax.ShapeDtypeStruct((NCORES, N, 128), jnp.float32),
          jax.ShapeDtypeStruct((EP,), jnp.float32),
          jax.ShapeDtypeStruct((EP,), jnp.float32),
      ),
      mesh=mesh,
      compiler_params=pltpu.CompilerParams(use_tc_tiling_on_sc=False),
      scratch_types=[
          pltpu.VMEM_SHARED((DN, 128), jnp.float32),
          pltpu.VMEM_SHARED((DN,), jnp.float32),
          pltpu.VMEM_SHARED((DN,), jnp.float32),
          pltpu.VMEM((16, 128), jnp.float32),
          pltpu.VMEM((DROWS,), jnp.float32),
          pltpu.VMEM((CH_D,), jnp.float32),
          pltpu.VMEM((CH_D,), jnp.float32),
      ] + [pltpu.VMEM((SUB,), jnp.int32)] * 8
        + [pltpu.VMEM((CH_P,), jnp.int32)] * 4
        + [pltpu.VMEM((CH_P,), jnp.float32)] * 14
        + [pltpu.VMEM((CH_P, 128), jnp.float32)] * 2
        + sems,
      name="sc_deg_norm_pass0",
  )


def _make_sc_pass2():
  mesh = plsc.VectorSubcoreMesh(
      core_axis_name="c", subcore_axis_name="s",
      num_cores=NCORES, num_subcores=NSUB)

  def body(src_hbm, dst_hbm, normc_hbm, norma_hbm, g_hbm, p_hbm,
           acc_sh, zrow,
           ps0, ps1, pd0, pd1, nc0, nc1, na0, na1,
           rows0, rows1,
           sem_z, sem_in0, sem_in1, sem_r0, sem_r1, sem_o0, sem_o1):
    c = lax.axis_index("c")
    s = lax.axis_index("s")

    _zero_acc(acc_sh, zrow, s, sem_z)
    plsc.subcore_barrier()

    ebase = (c * NSUB + s) * E_PASS
    sets = [
        dict(ps=ps0, pd=pd0, nc=nc0, na=na0, rows=rows0,
             sem_in=sem_in0, sem_r=sem_r0, sem_o=sem_o0),
        dict(ps=ps1, pd=pd1, nc=nc1, na=na1, rows=rows1,
             sem_in=sem_in1, sem_r=sem_r1, sem_o=sem_o1),
    ]

    def fire_loads(off, S):
      return [
          pltpu.async_copy(src_hbm.at[pl.ds(off, CH_P)], S['ps'], S['sem_in']),
          pltpu.async_copy(dst_hbm.at[pl.ds(off, CH_P)], S['pd'], S['sem_in']),
          pltpu.async_copy(normc_hbm.at[pl.ds(off, CH_P)], S['nc'],
                           S['sem_in']),
          pltpu.async_copy(norma_hbm.at[pl.ds(off, CH_P)], S['na'],
                           S['sem_in']),
      ]

    @pl.loop(0, PASS_PAIRS)
    def _(t):
      off0 = pl.multiple_of(ebase + t * (2 * CH_P), CH_P)
      off1 = pl.multiple_of(ebase + t * (2 * CH_P) + CH_P, CH_P)
      S0, S1 = sets
      l0 = fire_loads(off0, S0)
      l1 = fire_loads(off1, S1)
      _drain(l0)
      r0 = [pltpu.async_copy(g_hbm.at[S0['ps']], S0['rows'], S0['sem_r'])]
      _drain(l1)
      r1 = [pltpu.async_copy(g_hbm.at[S1['ps']], S1['rows'], S1['sem_r'])]
      _drain(r0)
      _scale_rows(S0['rows'], S0['nc'], S0['na'])
      s0 = [pltpu.async_copy(S0['rows'], acc_sh.at[S0['pd']],
                             S0['sem_o'], add=True)]
      _drain(r1)
      _scale_rows(S1['rows'], S1['nc'], S1['na'])
      s1 = [pltpu.async_copy(S1['rows'], acc_sh.at[S1['pd']],
                             S1['sem_o'], add=True)]
      _drain(s0)
      _drain(s1)

    plsc.subcore_barrier()
    rbase = s * DROWS
    @pl.when(s < NSUB - 1)
    def _():
      pltpu.sync_copy(acc_sh.at[pl.ds(rbase, DROWS), :],
                      p_hbm.at[c, pl.ds(rbase, DROWS), :])
    @pl.when(s == NSUB - 1)
    def _():
      pltpu.sync_copy(acc_sh.at[pl.ds(rbase, N - (NSUB - 1) * DROWS), :],
                      p_hbm.at[c, pl.ds(rbase, N - (NSUB - 1) * DROWS), :])

  return pl.kernel(
      body,
      out_type=jax.ShapeDtypeStruct((NCORES, N, 128), jnp.float32),
      mesh=mesh,
      compiler_params=pltpu.CompilerParams(use_tc_tiling_on_sc=False),
      scratch_types=[
          pltpu.VMEM_SHARED((DN, 128), jnp.float32),
          pltpu.VMEM((16, 128), jnp.float32),
      ] + [pltpu.VMEM((CH_P,), jnp.int32)] * 4
        + [pltpu.VMEM((CH_P,), jnp.float32)] * 4
        + [pltpu.VMEM((CH_P, 128), jnp.float32)] * 2
        + [pltpu.SemaphoreType.DMA] * 7,
      name="sc_pass1",
  )


_sc_pass1 = _make_sc_pass1()
_sc_pass2 = _make_sc_pass2()


RB = 1000
GRID = N // RB


def _dotT(a, w):
  return lax.dot_general(a, w, (((1,), (1,)), ((), ())),
                         preferred_element_type=jnp.float32)


def _dot(a, w):
  return lax.dot_general(a, w, (((1,), (0,)), ((), ())),
                         preferred_element_type=jnp.float32)


def _tc1_body(x_r, pxW_r, pxb_r, phW_r, phb_r, aiW_r, g_r, xp_r):
  xb = x_r[...]
  xp = _dotT(xb, pxW_r[...]) + pxb_r[...]
  hp = _dotT(xb, phW_r[...]) + phb_r[...]
  hh = _dot(xp, aiW_r[...])
  g_r[...] = jnp.concatenate([hp, hh], axis=1)
  xp_r[...] = xp


def _tc2_body(x_r, g0_r, xp0_r, p_r,
              cl0_r, cl1_r, cb_r, arW_r, ab_r, lW_r, lb_r,
              pxW_r, pxb_r, phW_r, phb_r, aiW_r,
              g1_r, xp1_r):
  p = p_r[0] + p_r[1]
  tx1 = p[:, :64]
  agg = p[:, 64:]
  hp0 = g0_r[:, :64]
  xp0 = xp0_r[...]
  o1 = _dotT(hp0, cl0_r[...]) + _dotT(tx1, cl1_r[...]) + cb_r[...]
  o1 = jnp.where(o1 >= 0, o1, 0.01 * o1)
  o2 = agg + _dot(xp0, arW_r[...]) + ab_r[...]
  o2 = jnp.maximum(o2, 0.0)
  o3 = _dotT(o1 + o2, lW_r[...]) + lb_r[...]
  xp1 = _dotT(o3, pxW_r[...]) + pxb_r[...]
  hp1 = _dotT(x_r[...], phW_r[...]) + phb_r[...]
  hh1 = _dot(xp1, aiW_r[...])
  g1_r[...] = jnp.concatenate([hp1, hh1], axis=1)
  xp1_r[...] = xp1


def _tc3_body(g1_r, xp1_r, p_r,
              cl0_r, cl1_r, cb_r, arW_r, ab_r, lW_r, lb_r,
              clsW_r, clsb_r, out_r):
  p = p_r[0] + p_r[1]
  tx1 = p[:, :64]
  agg = p[:, 64:]
  hp1 = g1_r[:, :64]
  o1 = _dotT(hp1, cl0_r[...]) + _dotT(tx1, cl1_r[...]) + cb_r[...]
  o1 = jnp.where(o1 >= 0, o1, 0.01 * o1)
  o2 = agg + _dot(xp1_r[...], arW_r[...]) + ab_r[...]
  o2 = jnp.maximum(o2, 0.0)
  o3 = _dotT(o1 + o2, lW_r[...]) + lb_r[...]
  logits = _dotT(o3, clsW_r[...]) + clsb_r[...]
  m = jnp.max(logits, axis=1, keepdims=True)
  sh = logits - m
  out_r[...] = sh - jnp.log(jnp.sum(jnp.exp(sh), axis=1, keepdims=True))


def _full(shape):
  return pl.BlockSpec(shape, lambda i: (0,) * len(shape))


def _rows(shape):
  return pl.BlockSpec(shape, lambda i: (i,) + (0,) * (len(shape) - 1))


def kernel(x, edge_index, edge_weight,
           c0_pre_h_W, c0_pre_h_b, c0_pre_x_W, c0_pre_x_b,
           c0_cheb_lin0_W, c0_cheb_lin1_W, c0_cheb_b,
           c0_arma_init_W, c0_arma_root_W, c0_arma_b,
           c0_lin_W, c0_lin_b,
           c1_pre_h_W, c1_pre_h_b, c1_pre_x_W, c1_pre_x_b,
           c1_cheb_lin0_W, c1_cheb_lin1_W, c1_cheb_b,
           c1_arma_init_W, c1_arma_root_W, c1_arma_b,
           c1_lin_W, c1_lin_b,
           cls_W, cls_b):
  r2 = lambda b: b.reshape(1, -1)

  pad_idx = (jnp.arange(PAD, dtype=jnp.int32) * 997) % N
  src = jnp.concatenate([edge_index[0], pad_idx])
  dst = jnp.concatenate([edge_index[1], pad_idx])
  ew = jnp.concatenate([edge_weight, jnp.zeros((PAD,), jnp.float32)])

  g0, xp0 = pl.pallas_call(
      _tc1_body,
      grid=(GRID,),
      in_specs=[_rows((RB, 128)), _full((64, 128)), _full((1, 64)),
                _full((64, 128)), _full((1, 64)), _full((64, 64))],
      out_specs=[_rows((RB, 128)), _rows((RB, 64))],
      out_shape=[jax.ShapeDtypeStruct((N, 128), jnp.float32),
                 jax.ShapeDtypeStruct((N, 64), jnp.float32)],
  )(x, c0_pre_x_W, r2(c0_pre_x_b), c0_pre_h_W, r2(c0_pre_h_b),
    c0_arma_init_W)

  p0, normc, norma = _sc_pass1(src, dst, ew, g0)

  g1, xp1 = pl.pallas_call(
      _tc2_body,
      grid=(GRID,),
      in_specs=[_rows((RB, 128)), _rows((RB, 128)), _rows((RB, 64)),
                pl.BlockSpec((2, RB, 128), lambda i: (0, i, 0)),
                _full((64, 64)), _full((64, 64)), _full((1, 64)),
                _full((64, 64)), _full((1, 64)),
                _full((64, 64)), _full((1, 64)),
                _full((64, 64)), _full((1, 64)),
                _full((64, 128)), _full((1, 64)), _full((64, 64))],
      out_specs=[_rows((RB, 128)), _rows((RB, 64))],
      out_shape=[jax.ShapeDtypeStruct((N, 128), jnp.float32),
                 jax.ShapeDtypeStruct((N, 64), jnp.float32)],
  )(x, g0, xp0, p0,
    c0_cheb_lin0_W, c0_cheb_lin1_W, r2(c0_cheb_b),
    c0_arma_root_W, r2(c0_arma_b), c0_lin_W, r2(c0_lin_b),
    c1_pre_x_W, r2(c1_pre_x_b), c1_pre_h_W, r2(c1_pre_h_b),
    c1_arma_init_W)

  p1 = _sc_pass2(src, dst, normc, norma, g1)

  out = pl.pallas_call(
      _tc3_body,
      grid=(GRID,),
      in_specs=[_rows((RB, 128)), _rows((RB, 64)),
                pl.BlockSpec((2, RB, 128), lambda i: (0, i, 0)),
                _full((64, 64)), _full((64, 64)), _full((1, 64)),
                _full((64, 64)), _full((1, 64)),
                _full((64, 64)), _full((1, 64)),
                _full((32, 64)), _full((1, 32))],
      out_specs=_rows((RB, NC_CLS)),
      out_shape=jax.ShapeDtypeStruct((N, NC_CLS), jnp.float32),
  )(g1, xp1, p1,
    c1_cheb_lin0_W, c1_cheb_lin1_W, r2(c1_cheb_b),
    c1_arma_root_W, r2(c1_arma_b), c1_lin_W, r2(c1_lin_b),
    cls_W, r2(cls_b))

  return out


# speedup vs baseline: 33.0298x; 1.1610x over previous
import jax
import jax.numpy as jnp
from jax import lax
from jax.experimental import pallas as pl
from jax.experimental.pallas import tpu as pltpu
from jax.experimental.pallas import tpu_sc as plsc

N = 10000
NC_CLS = 32
E = 320000

NCORES = 2
NSUB = 16
NW = NCORES * NSUB

SUB = 128
CH_P = 128
CH_D = 512
NSC_D = CH_D // SUB

EP = 327680
PAD = EP - E
E_DEG = EP // NSUB
E_PASS = EP // NW
DEG_CHUNKS = E_DEG // CH_D
PASS_PAIRS = E_PASS // (2 * CH_P)

DN = 10240
DROWS = DN // NSUB


def _newton_rsqrt(v):
  b = lax.bitcast_convert_type(v, jnp.int32)
  h = jnp.int32(0x5F3759DF) - (b >> 1)
  y = lax.bitcast_convert_type(h, jnp.float32)
  for _ in range(4):
    y = y * (1.5 - 0.5 * v * y * y)
  return y


def _zero_vmem2d(ref, rows, cols):
  z = jnp.zeros((16,), jnp.float32)
  for r in range(rows):
    for c0 in range(cols // 16):
      ref[r, pl.ds(c0 * 16, 16)] = z


def _zero_acc(acc_sh, zrow, s, sem):
  _zero_vmem2d(zrow, 16, 128)
  rbase = s * DROWS
  descs = []
  for j in range(DROWS // 16):
    descs.append(pltpu.async_copy(
        zrow, acc_sh.at[pl.ds(rbase + j * 16, 16), :], sem))
  _drain(descs)


def _drain(descs):
  for d in descs:
    d.wait()


def _scale_rows(rows_ref, nc_ref, na_ref):
  @pl.loop(0, CH_P // 16)
  def _(g):
    base = pl.multiple_of(g * 16, 16)
    ncv = nc_ref[pl.ds(base, 16)]
    nav = na_ref[pl.ds(base, 16)]
    for i in range(16):
      e = base + i
      ncs = ncv[i]
      nas = nav[i]
      for f in range(4):
        rows_ref[e, pl.ds(f * 16, 16)] = rows_ref[e, pl.ds(f * 16, 16)] * ncs
      for f in range(4, 8):
        rows_ref[e, pl.ds(f * 16, 16)] = rows_ref[e, pl.ds(f * 16, 16)] * nas


def _make_sc_pass1():
  mesh = plsc.VectorSubcoreMesh(
      core_axis_name="c", subcore_axis_name="s",
      num_cores=NCORES, num_subcores=NSUB)

  def body(src_hbm, dst_hbm, ew_hbm, g_hbm, p_hbm, normc_hbm, norma_hbm,
           acc_sh, degc_sh, dega_sh,
           zrow, dwork, wdeg, wcdeg,
           ds0, ds1, ds2, ds3, dd0, dd1, dd2, dd3,
           ps0, ps1, pd0, pd1, pdx0, pdx1, wp0, wp1, nc0, nc1, na0, na1,
           dcs0, dcs1, dcd0, dcd1, das0, das1, dad0, dad1,
           rows0, rows1,
           sem_z, sem_dl, sem_dsc,
           sem_in0, sem_in1, sem_g0, sem_g1, sem_r0, sem_r1,
           sem_o0, sem_o1, sem_n0, sem_n1):
    c = lax.axis_index("c")
    s = lax.axis_index("s")
    dsrc = [ds0, ds1, ds2, ds3]
    ddst = [dd0, dd1, dd2, dd3]

    _zero_acc(acc_sh, zrow, s, sem_z)
    zflat = zrow.at[0]
    descs = []
    for j in range(DROWS // 128):
      descs.append(pltpu.async_copy(
          zflat, degc_sh.at[pl.ds(s * DROWS + j * 128, 128)], sem_z))
      descs.append(pltpu.async_copy(
          zflat, dega_sh.at[pl.ds(s * DROWS + j * 128, 128)], sem_z))
    _drain(descs)
    plsc.subcore_barrier()

    dbase = s * E_DEG
    dsets = [
        dict(sr=[ds0, ds1], dr=[dd0, dd1], woff=0, sem_l=sem_dl,
             sem_s=sem_dsc),
        dict(sr=[ds2, ds3], dr=[dd2, dd3], woff=CH_D // 2, sem_l=sem_in0,
             sem_s=sem_o0),
    ]

    def deg_loads(off, S):
      descs = []
      for k in range(2):
        descs.append(pltpu.async_copy(
            src_hbm.at[pl.ds(off + k * SUB, SUB)], S['sr'][k], S['sem_l']))
        descs.append(pltpu.async_copy(
            dst_hbm.at[pl.ds(off + k * SUB, SUB)], S['dr'][k], S['sem_l']))
      descs.append(pltpu.async_copy(
          ew_hbm.at[pl.ds(off, CH_D // 2)],
          wdeg.at[pl.ds(S['woff'], CH_D // 2)], S['sem_l']))
      return descs

    def deg_compute_scatter(S):
      for g in range(CH_D // 2 // 16):
        k, col = divmod(g * 16, SUB)
        sl = pl.ds(S['woff'] + g * 16, 16)
        sv = S['sr'][k][pl.ds(col, 16)]
        dv = S['dr'][k][pl.ds(col, 16)]
        wcdeg[sl] = jnp.where(sv == dv, 0.0, wdeg[sl])
      descs = []
      for k in range(2):
        descs.append(pltpu.async_copy(
            wcdeg.at[pl.ds(S['woff'] + k * SUB, SUB)], degc_sh.at[S['sr'][k]],
            S['sem_s'], add=True))
        descs.append(pltpu.async_copy(
            wdeg.at[pl.ds(S['woff'] + k * SUB, SUB)], dega_sh.at[S['dr'][k]],
            S['sem_s'], add=True))
      return descs

    @pl.loop(0, DEG_CHUNKS)
    def _(j):
      off = pl.multiple_of(dbase + j * CH_D, CH_D)
      A, B = dsets
      lA = deg_loads(off, A)
      lB = deg_loads(off + CH_D // 2, B)
      _drain(lA)
      sA = deg_compute_scatter(A)
      _drain(lB)
      sB = deg_compute_scatter(B)
      _drain(sA)
      _drain(sB)

    plsc.subcore_barrier()

    rbase = s * DROWS
    for deg_sh in (degc_sh, dega_sh):
      pltpu.sync_copy(deg_sh.at[pl.ds(rbase, DROWS)], dwork)
      @pl.loop(0, DROWS // 16)
      def _(i):
        sl = pl.ds(pl.multiple_of(i * 16, 16), 16)
        v = dwork[sl]
        dwork[sl] = jnp.where(v > 0, _newton_rsqrt(v), 0.0)
      pltpu.sync_copy(dwork, deg_sh.at[pl.ds(rbase, DROWS)])
    plsc.subcore_barrier()

    ebase = (c * NSUB + s) * E_PASS
    sets = [
        dict(ps=ps0, pd=pd0, pdx=pdx0, wp=wp0, nc=nc0, na=na0, dcs=dcs0,
             dcd=dcd0, das=das0, dad=dad0, rows=rows0, sem_in=sem_in0,
             sem_g=sem_g0, sem_r=sem_r0, sem_o=sem_o0, sem_n=sem_n0),
        dict(ps=ps1, pd=pd1, pdx=pdx1, wp=wp1, nc=nc1, na=na1, dcs=dcs1,
             dcd=dcd1, das=das1, dad=dad1, rows=rows1, sem_in=sem_in1,
             sem_g=sem_g1, sem_r=sem_r1, sem_o=sem_o1, sem_n=sem_n1),
    ]

    def fire_loads(off, S):
      return [
          pltpu.async_copy(src_hbm.at[pl.ds(off, CH_P)], S['ps'], S['sem_in']),
          pltpu.async_copy(dst_hbm.at[pl.ds(off, CH_P)], S['pd'], S['sem_in']),
          pltpu.async_copy(ew_hbm.at[pl.ds(off, CH_P)], S['wp'], S['sem_in']),
      ]

    def fire_gathers(S):
      g = [
          pltpu.async_copy(degc_sh.at[S['ps']], S['dcs'], S['sem_g']),
          pltpu.async_copy(degc_sh.at[S['pd']], S['dcd'], S['sem_g']),
          pltpu.async_copy(dega_sh.at[S['ps']], S['das'], S['sem_g']),
          pltpu.async_copy(dega_sh.at[S['pd']], S['dad'], S['sem_g']),
      ]
      r = [pltpu.async_copy(g_hbm.at[S['ps']], S['rows'], S['sem_r'])]
      return g, r

    def compute_and_out(off, S):
      for g in range(CH_P // 16):
        sl = pl.ds(g * 16, 16)
        sv = S['ps'][sl]
        dv = S['pd'][sl]
        wv = S['wp'][sl]
        wc = jnp.where(sv == dv, 0.0, wv)
        S['nc'][sl] = -(S['dcs'][sl] * wc * S['dcd'][sl])
        S['na'][sl] = S['das'][sl] * wv * S['dad'][sl]
      nw = [
          pltpu.async_copy(S['nc'], normc_hbm.at[pl.ds(off, CH_P)], S['sem_n']),
          pltpu.async_copy(S['na'], norma_hbm.at[pl.ds(off, CH_P)], S['sem_n']),
      ]
      return nw

    def fire_scatter(S):
      return [pltpu.async_copy(S['rows'], acc_sh.at[S['pdx']],
                               S['sem_o'], add=True)]

    def wait_prev_scatter(S, t):
      @pl.when(t > 0)
      def _():
        pltpu.make_async_copy(S['rows'], acc_sh.at[S['pdx']],
                              S['sem_o']).wait()

    def wait_prev_nw(S, t):
      @pl.when(t > 0)
      def _():
        pltpu.make_async_copy(S['nc'], normc_hbm.at[pl.ds(0, CH_P)],
                              S['sem_n']).wait()
        pltpu.make_async_copy(S['na'], norma_hbm.at[pl.ds(0, CH_P)],
                              S['sem_n']).wait()

    def snap_idx(S):
      for g in range(CH_P // 16):
        sl = pl.ds(g * 16, 16)
        S['pdx'][sl] = S['pd'][sl]

    @pl.loop(0, PASS_PAIRS)
    def _(t):
      off0 = pl.multiple_of(ebase + t * (2 * CH_P), CH_P)
      off1 = pl.multiple_of(ebase + t * (2 * CH_P) + CH_P, CH_P)
      S0, S1 = sets
      l0 = fire_loads(off0, S0)
      l1 = fire_loads(off1, S1)
      _drain(l0)
      wait_prev_scatter(S0, t)
      g0, r0 = fire_gathers(S0)
      _drain(l1)
      wait_prev_scatter(S1, t)
      g1, r1 = fire_gathers(S1)
      _drain(g0)
      wait_prev_nw(S0, t)
      nw0 = compute_and_out(off0, S0)
      _drain(r0)
      _scale_rows(S0['rows'], S0['nc'], S0['na'])
      snap_idx(S0)
      fire_scatter(S0)
      _drain(g1)
      wait_prev_nw(S1, t)
      nw1 = compute_and_out(off1, S1)
      _drain(r1)
      _scale_rows(S1['rows'], S1['nc'], S1['na'])
      snap_idx(S1)
      fire_scatter(S1)

    for S in sets:
      pltpu.make_async_copy(S['rows'], acc_sh.at[S['pdx']], S['sem_o']).wait()
      pltpu.make_async_copy(S['nc'], normc_hbm.at[pl.ds(0, CH_P)],
                            S['sem_n']).wait()
      pltpu.make_async_copy(S['na'], norma_hbm.at[pl.ds(0, CH_P)],
                            S['sem_n']).wait()
    plsc.subcore_barrier()

    @pl.when(s < NSUB - 1)
    def _():
      pltpu.sync_copy(acc_sh.at[pl.ds(rbase, DROWS), :],
                      p_hbm.at[c, pl.ds(rbase, DROWS), :])
    @pl.when(s == NSUB - 1)
    def _():
      pltpu.sync_copy(acc_sh.at[pl.ds(rbase, N - (NSUB - 1) * DROWS), :],
                      p_hbm.at[c, pl.ds(rbase, N - (NSUB - 1) * DROWS), :])

  sems = [pltpu.SemaphoreType.DMA] * 13
  return pl.kernel(
      body,
      out_type=(
          jax.ShapeDtypeStruct((NCORES, N, 128), jnp.float32),
          jax.ShapeDtypeStruct((EP,), jnp.float32),
          jax.ShapeDtypeStruct((EP,), jnp.float32),
      ),
      mesh=mesh,
      compiler_params=pltpu.CompilerParams(use_tc_tiling_on_sc=False),
      scratch_types=[
          pltpu.VMEM_SHARED((DN, 128), jnp.float32),
          pltpu.VMEM_SHARED((DN,), jnp.float32),
          pltpu.VMEM_SHARED((DN,), jnp.float32),
          pltpu.VMEM((16, 128), jnp.float32),
          pltpu.VMEM((DROWS,), jnp.float32),
          pltpu.VMEM((CH_D,), jnp.float32),
          pltpu.VMEM((CH_D,), jnp.float32),
      ] + [pltpu.VMEM((SUB,), jnp.int32)] * 8
        + [pltpu.VMEM((CH_P,), jnp.int32)] * 6
        + [pltpu.VMEM((CH_P,), jnp.float32)] * 14
        + [pltpu.VMEM((CH_P, 128), jnp.float32)] * 2
        + sems,
      name="sc_deg_norm_pass0",
  )


def _make_sc_pass2():
  mesh = plsc.VectorSubcoreMesh(
      core_axis_name="c", subcore_axis_name="s",
      num_cores=NCORES, num_subcores=NSUB)

  def body(src_hbm, dst_hbm, normc_hbm, norma_hbm, g_hbm, p_hbm,
           acc_sh, zrow,
           ps0, ps1, pd0, pd1, pdx0, pdx1, nc0, nc1, na0, na1,
           rows0, rows1,
           sem_z, sem_in0, sem_in1, sem_r0, sem_r1, sem_o0, sem_o1):
    c = lax.axis_index("c")
    s = lax.axis_index("s")

    _zero_acc(acc_sh, zrow, s, sem_z)
    plsc.subcore_barrier()

    ebase = (c * NSUB + s) * E_PASS
    sets = [
        dict(ps=ps0, pd=pd0, pdx=pdx0, nc=nc0, na=na0, rows=rows0,
             sem_in=sem_in0, sem_r=sem_r0, sem_o=sem_o0),
        dict(ps=ps1, pd=pd1, pdx=pdx1, nc=nc1, na=na1, rows=rows1,
             sem_in=sem_in1, sem_r=sem_r1, sem_o=sem_o1),
    ]

    def fire_loads(off, S):
      return [
          pltpu.async_copy(src_hbm.at[pl.ds(off, CH_P)], S['ps'], S['sem_in']),
          pltpu.async_copy(dst_hbm.at[pl.ds(off, CH_P)], S['pd'], S['sem_in']),
          pltpu.async_copy(normc_hbm.at[pl.ds(off, CH_P)], S['nc'],
                           S['sem_in']),
          pltpu.async_copy(norma_hbm.at[pl.ds(off, CH_P)], S['na'],
                           S['sem_in']),
      ]

    def wait_prev(S, t):
      @pl.when(t > 0)
      def _():
        pltpu.make_async_copy(S['rows'], acc_sh.at[S['pdx']],
                              S['sem_o']).wait()

    def snap_idx(S):
      for g in range(CH_P // 16):
        sl = pl.ds(g * 16, 16)
        S['pdx'][sl] = S['pd'][sl]

    @pl.loop(0, PASS_PAIRS)
    def _(t):
      off0 = pl.multiple_of(ebase + t * (2 * CH_P), CH_P)
      off1 = pl.multiple_of(ebase + t * (2 * CH_P) + CH_P, CH_P)
      S0, S1 = sets
      l0 = fire_loads(off0, S0)
      l1 = fire_loads(off1, S1)
      _drain(l0)
      wait_prev(S0, t)
      r0 = [pltpu.async_copy(g_hbm.at[S0['ps']], S0['rows'], S0['sem_r'])]
      _drain(l1)
      wait_prev(S1, t)
      r1 = [pltpu.async_copy(g_hbm.at[S1['ps']], S1['rows'], S1['sem_r'])]
      _drain(r0)
      _scale_rows(S0['rows'], S0['nc'], S0['na'])
      snap_idx(S0)
      pltpu.async_copy(S0['rows'], acc_sh.at[S0['pdx']], S0['sem_o'], add=True)
      _drain(r1)
      _scale_rows(S1['rows'], S1['nc'], S1['na'])
      snap_idx(S1)
      pltpu.async_copy(S1['rows'], acc_sh.at[S1['pdx']], S1['sem_o'], add=True)

    for S in sets:
      pltpu.make_async_copy(S['rows'], acc_sh.at[S['pdx']], S['sem_o']).wait()
    plsc.subcore_barrier()
    rbase = s * DROWS
    @pl.when(s < NSUB - 1)
    def _():
      pltpu.sync_copy(acc_sh.at[pl.ds(rbase, DROWS), :],
                      p_hbm.at[c, pl.ds(rbase, DROWS), :])
    @pl.when(s == NSUB - 1)
    def _():
      pltpu.sync_copy(acc_sh.at[pl.ds(rbase, N - (NSUB - 1) * DROWS), :],
                      p_hbm.at[c, pl.ds(rbase, N - (NSUB - 1) * DROWS), :])

  return pl.kernel(
      body,
      out_type=jax.ShapeDtypeStruct((NCORES, N, 128), jnp.float32),
      mesh=mesh,
      compiler_params=pltpu.CompilerParams(use_tc_tiling_on_sc=False),
      scratch_types=[
          pltpu.VMEM_SHARED((DN, 128), jnp.float32),
          pltpu.VMEM((16, 128), jnp.float32),
      ] + [pltpu.VMEM((CH_P,), jnp.int32)] * 6
        + [pltpu.VMEM((CH_P,), jnp.float32)] * 4
        + [pltpu.VMEM((CH_P, 128), jnp.float32)] * 2
        + [pltpu.SemaphoreType.DMA] * 7,
      name="sc_pass1",
  )


_sc_pass1 = _make_sc_pass1()
_sc_pass2 = _make_sc_pass2()


RB = 1000
GRID = N // RB


def _dotT(a, w):
  return lax.dot_general(a, w, (((1,), (1,)), ((), ())),
                         preferred_element_type=jnp.float32)


def _dot(a, w):
  return lax.dot_general(a, w, (((1,), (0,)), ((), ())),
                         preferred_element_type=jnp.float32)


def _tc1_body(x_r, pxW_r, pxb_r, phW_r, phb_r, aiW_r, g_r, xp_r):
  xb = x_r[...]
  xp = _dotT(xb, pxW_r[...]) + pxb_r[...]
  hp = _dotT(xb, phW_r[...]) + phb_r[...]
  hh = _dot(xp, aiW_r[...])
  g_r[...] = jnp.concatenate([hp, hh], axis=1)
  xp_r[...] = xp


def _tc2_body(x_r, g0_r, xp0_r, p_r,
              cl0_r, cl1_r, cb_r, arW_r, ab_r, lW_r, lb_r,
              pxW_r, pxb_r, phW_r, phb_r, aiW_r,
              g1_r, xp1_r):
  p = p_r[0] + p_r[1]
  tx1 = p[:, :64]
  agg = p[:, 64:]
  hp0 = g0_r[:, :64]
  xp0 = xp0_r[...]
  o1 = _dotT(hp0, cl0_r[...]) + _dotT(tx1, cl1_r[...]) + cb_r[...]
  o1 = jnp.where(o1 >= 0, o1, 0.01 * o1)
  o2 = agg + _dot(xp0, arW_r[...]) + ab_r[...]
  o2 = jnp.maximum(o2, 0.0)
  o3 = _dotT(o1 + o2, lW_r[...]) + lb_r[...]
  xp1 = _dotT(o3, pxW_r[...]) + pxb_r[...]
  hp1 = _dotT(x_r[...], phW_r[...]) + phb_r[...]
  hh1 = _dot(xp1, aiW_r[...])
  g1_r[...] = jnp.concatenate([hp1, hh1], axis=1)
  xp1_r[...] = xp1


def _tc3_body(g1_r, xp1_r, p_r,
              cl0_r, cl1_r, cb_r, arW_r, ab_r, lW_r, lb_r,
              clsW_r, clsb_r, out_r):
  p = p_r[0] + p_r[1]
  tx1 = p[:, :64]
  agg = p[:, 64:]
  hp1 = g1_r[:, :64]
  o1 = _dotT(hp1, cl0_r[...]) + _dotT(tx1, cl1_r[...]) + cb_r[...]
  o1 = jnp.where(o1 >= 0, o1, 0.01 * o1)
  o2 = agg + _dot(xp1_r[...], arW_r[...]) + ab_r[...]
  o2 = jnp.maximum(o2, 0.0)
  o3 = _dotT(o1 + o2, lW_r[...]) + lb_r[...]
  logits = _dotT(o3, clsW_r[...]) + clsb_r[...]
  m = jnp.max(logits, axis=1, keepdims=True)
  sh = logits - m
  out_r[...] = sh - jnp.log(jnp.sum(jnp.exp(sh), axis=1, keepdims=True))


def _full(shape):
  return pl.BlockSpec(shape, lambda i: (0,) * len(shape))


def _rows(shape):
  return pl.BlockSpec(shape, lambda i: (i,) + (0,) * (len(shape) - 1))


def kernel(x, edge_index, edge_weight,
           c0_pre_h_W, c0_pre_h_b, c0_pre_x_W, c0_pre_x_b,
           c0_cheb_lin0_W, c0_cheb_lin1_W, c0_cheb_b,
           c0_arma_init_W, c0_arma_root_W, c0_arma_b,
           c0_lin_W, c0_lin_b,
           c1_pre_h_W, c1_pre_h_b, c1_pre_x_W, c1_pre_x_b,
           c1_cheb_lin0_W, c1_cheb_lin1_W, c1_cheb_b,
           c1_arma_init_W, c1_arma_root_W, c1_arma_b,
           c1_lin_W, c1_lin_b,
           cls_W, cls_b):
  r2 = lambda b: b.reshape(1, -1)

  pad_idx = (jnp.arange(PAD, dtype=jnp.int32) * 997) % N
  src = jnp.concatenate([edge_index[0], pad_idx])
  dst = jnp.concatenate([edge_index[1], pad_idx])
  ew = jnp.concatenate([edge_weight, jnp.zeros((PAD,), jnp.float32)])

  g0, xp0 = pl.pallas_call(
      _tc1_body,
      grid=(GRID,),
      in_specs=[_rows((RB, 128)), _full((64, 128)), _full((1, 64)),
                _full((64, 128)), _full((1, 64)), _full((64, 64))],
      out_specs=[_rows((RB, 128)), _rows((RB, 64))],
      out_shape=[jax.ShapeDtypeStruct((N, 128), jnp.float32),
                 jax.ShapeDtypeStruct((N, 64), jnp.float32)],
  )(x, c0_pre_x_W, r2(c0_pre_x_b), c0_pre_h_W, r2(c0_pre_h_b),
    c0_arma_init_W)

  p0, normc, norma = _sc_pass1(src, dst, ew, g0)

  g1, xp1 = pl.pallas_call(
      _tc2_body,
      grid=(GRID,),
      in_specs=[_rows((RB, 128)), _rows((RB, 128)), _rows((RB, 64)),
                pl.BlockSpec((2, RB, 128), lambda i: (0, i, 0)),
                _full((64, 64)), _full((64, 64)), _full((1, 64)),
                _full((64, 64)), _full((1, 64)),
                _full((64, 64)), _full((1, 64)),
                _full((64, 64)), _full((1, 64)),
                _full((64, 128)), _full((1, 64)), _full((64, 64))],
      out_specs=[_rows((RB, 128)), _rows((RB, 64))],
      out_shape=[jax.ShapeDtypeStruct((N, 128), jnp.float32),
                 jax.ShapeDtypeStruct((N, 64), jnp.float32)],
  )(x, g0, xp0, p0,
    c0_cheb_lin0_W, c0_cheb_lin1_W, r2(c0_cheb_b),
    c0_arma_root_W, r2(c0_arma_b), c0_lin_W, r2(c0_lin_b),
    c1_pre_x_W, r2(c1_pre_x_b), c1_pre_h_W, r2(c1_pre_h_b),
    c1_arma_init_W)

  p1 = _sc_pass2(src, dst, normc, norma, g1)

  out = pl.pallas_call(
      _tc3_body,
      grid=(GRID,),
      in_specs=[_rows((RB, 128)), _rows((RB, 64)),
                pl.BlockSpec((2, RB, 128), lambda i: (0, i, 0)),
                _full((64, 64)), _full((64, 64)), _full((1, 64)),
                _full((64, 64)), _full((1, 64)),
                _full((64, 64)), _full((1, 64)),
                _full((32, 64)), _full((1, 32))],
      out_specs=_rows((RB, NC_CLS)),
      out_shape=jax.ShapeDtypeStruct((N, NC_CLS), jnp.float32),
  )(g1, xp1, p1,
    c1_cheb_lin0_W, c1_cheb_lin1_W, r2(c1_cheb_b),
    c1_arma_root_W, r2(c1_arma_b), c1_lin_W, r2(c1_lin_b),
    cls_W, r2(cls_b))

  return out


# speedup vs baseline: 33.6378x; 1.0184x over previous
import jax
import jax.numpy as jnp
from jax import lax
from jax.experimental import pallas as pl
from jax.experimental.pallas import tpu as pltpu
from jax.experimental.pallas import tpu_sc as plsc

N = 10000
NC_CLS = 32
E = 320000

NCORES = 2
NSUB = 16
NW = NCORES * NSUB

SUB = 128
CH_P = 128
CH_D = 512
NSC_D = CH_D // SUB

EP = 327680
PAD = EP - E
E_DEG = EP // NSUB
E_PASS = EP // NW
DEG_CHUNKS = E_DEG // CH_D
PASS_PAIRS = E_PASS // (2 * CH_P)

DN = 10240
DROWS = DN // NSUB


def _newton_rsqrt(v):
  b = lax.bitcast_convert_type(v, jnp.int32)
  h = jnp.int32(0x5F3759DF) - (b >> 1)
  y = lax.bitcast_convert_type(h, jnp.float32)
  for _ in range(4):
    y = y * (1.5 - 0.5 * v * y * y)
  return y


def _zero_vmem2d(ref, rows, cols):
  z = jnp.zeros((16,), jnp.float32)
  for r in range(rows):
    for c0 in range(cols // 16):
      ref[r, pl.ds(c0 * 16, 16)] = z


def _zero_acc(acc_sh, zrow, s, sem):
  _zero_vmem2d(zrow, 16, 128)
  rbase = s * DROWS
  descs = []
  for j in range(DROWS // 16):
    descs.append(pltpu.async_copy(
        zrow, acc_sh.at[pl.ds(rbase + j * 16, 16), :], sem))
  _drain(descs)


def _drain(descs):
  for d in descs:
    d.wait()


def _scale_rows(rows_ref, nc_ref, na_ref):
  @pl.loop(0, CH_P // 16)
  def _(g):
    base = pl.multiple_of(g * 16, 16)
    ncv = nc_ref[pl.ds(base, 16)]
    nav = na_ref[pl.ds(base, 16)]
    for i in range(16):
      e = base + i
      ncs = ncv[i]
      nas = nav[i]
      for f in range(4):
        rows_ref[e, pl.ds(f * 16, 16)] = rows_ref[e, pl.ds(f * 16, 16)] * ncs
      for f in range(4, 8):
        rows_ref[e, pl.ds(f * 16, 16)] = rows_ref[e, pl.ds(f * 16, 16)] * nas


def _make_sc_pass1():
  mesh = plsc.VectorSubcoreMesh(
      core_axis_name="c", subcore_axis_name="s",
      num_cores=NCORES, num_subcores=NSUB)

  def body(src_hbm, dst_hbm, ew_hbm, g_hbm, p_hbm, normc_hbm, norma_hbm,
           acc_sh, degc_sh, dega_sh,
           zrow, dwork, wdeg, wcdeg, wdegx, wcdegx,
           ds0, ds1, ds2, ds3, dd0, dd1, dd2, dd3,
           dsx0, dsx1, dsx2, dsx3, ddx0, ddx1, ddx2, ddx3,
           ps0, ps1, pd0, pd1, pdx0, pdx1, wp0, wp1, nc0, nc1, na0, na1,
           dcs0, dcs1, dcd0, dcd1, das0, das1, dad0, dad1,
           rows0, rows1,
           sem_z, sem_dl, sem_dsc,
           sem_in0, sem_in1, sem_g0, sem_g1, sem_r0, sem_r1,
           sem_o0, sem_o1, sem_n0, sem_n1):
    c = lax.axis_index("c")
    s = lax.axis_index("s")
    dsrc = [ds0, ds1, ds2, ds3]
    ddst = [dd0, dd1, dd2, dd3]

    _zero_acc(acc_sh, zrow, s, sem_z)
    zflat = zrow.at[0]
    descs = []
    for j in range(DROWS // 128):
      descs.append(pltpu.async_copy(
          zflat, degc_sh.at[pl.ds(s * DROWS + j * 128, 128)], sem_z))
      descs.append(pltpu.async_copy(
          zflat, dega_sh.at[pl.ds(s * DROWS + j * 128, 128)], sem_z))
    _drain(descs)
    plsc.subcore_barrier()

    dbase = s * E_DEG
    dsets = [
        dict(sr=[ds0, ds1], dr=[dd0, dd1], srx=[dsx0, dsx1],
             drx=[ddx0, ddx1], woff=0, sem_l=sem_dl, sem_s=sem_dsc),
        dict(sr=[ds2, ds3], dr=[dd2, dd3], srx=[dsx2, dsx3],
             drx=[ddx2, ddx3], woff=CH_D // 2, sem_l=sem_in0,
             sem_s=sem_o0),
    ]

    def deg_loads(off, S):
      descs = []
      for k in range(2):
        descs.append(pltpu.async_copy(
            src_hbm.at[pl.ds(off + k * SUB, SUB)], S['sr'][k], S['sem_l']))
        descs.append(pltpu.async_copy(
            dst_hbm.at[pl.ds(off + k * SUB, SUB)], S['dr'][k], S['sem_l']))
      descs.append(pltpu.async_copy(
          ew_hbm.at[pl.ds(off, CH_D // 2)],
          wdeg.at[pl.ds(S['woff'], CH_D // 2)], S['sem_l']))
      return descs

    def deg_compute_scatter(S):
      for g in range(CH_D // 2 // 16):
        k, col = divmod(g * 16, SUB)
        sl = pl.ds(S['woff'] + g * 16, 16)
        sv = S['sr'][k][pl.ds(col, 16)]
        dv = S['dr'][k][pl.ds(col, 16)]
        wcdeg[sl] = jnp.where(sv == dv, 0.0, wdeg[sl])
        S['srx'][k][pl.ds(col, 16)] = sv
        S['drx'][k][pl.ds(col, 16)] = dv
        wdegx[sl] = wdeg[sl]
        wcdegx[sl] = wcdeg[sl]
      for k in range(2):
        pltpu.async_copy(
            wcdegx.at[pl.ds(S['woff'] + k * SUB, SUB)],
            degc_sh.at[S['srx'][k]], S['sem_s'], add=True)
        pltpu.async_copy(
            wdegx.at[pl.ds(S['woff'] + k * SUB, SUB)],
            dega_sh.at[S['drx'][k]], S['sem_s'], add=True)

    def deg_wait_prev(S):
      for k in range(2):
        pltpu.make_async_copy(
            wcdegx.at[pl.ds(S['woff'] + k * SUB, SUB)],
            degc_sh.at[S['srx'][k]], S['sem_s']).wait()
        pltpu.make_async_copy(
            wdegx.at[pl.ds(S['woff'] + k * SUB, SUB)],
            dega_sh.at[S['drx'][k]], S['sem_s']).wait()

    @pl.loop(0, DEG_CHUNKS)
    def _(j):
      off = pl.multiple_of(dbase + j * CH_D, CH_D)
      A, B = dsets
      lA = deg_loads(off, A)
      lB = deg_loads(off + CH_D // 2, B)
      _drain(lA)
      @pl.when(j > 0)
      def _():
        deg_wait_prev(A)
      deg_compute_scatter(A)
      _drain(lB)
      @pl.when(j > 0)
      def _():
        deg_wait_prev(B)
      deg_compute_scatter(B)

    for S in dsets:
      deg_wait_prev(S)
    plsc.subcore_barrier()

    rbase = s * DROWS
    for deg_sh in (degc_sh, dega_sh):
      pltpu.sync_copy(deg_sh.at[pl.ds(rbase, DROWS)], dwork)
      @pl.loop(0, DROWS // 16)
      def _(i):
        sl = pl.ds(pl.multiple_of(i * 16, 16), 16)
        v = dwork[sl]
        dwork[sl] = jnp.where(v > 0, _newton_rsqrt(v), 0.0)
      pltpu.sync_copy(dwork, deg_sh.at[pl.ds(rbase, DROWS)])
    plsc.subcore_barrier()

    ebase = (c * NSUB + s) * E_PASS
    sets = [
        dict(ps=ps0, pd=pd0, pdx=pdx0, wp=wp0, nc=nc0, na=na0, dcs=dcs0,
             dcd=dcd0, das=das0, dad=dad0, rows=rows0, sem_in=sem_in0,
             sem_g=sem_g0, sem_r=sem_r0, sem_o=sem_o0, sem_n=sem_n0),
        dict(ps=ps1, pd=pd1, pdx=pdx1, wp=wp1, nc=nc1, na=na1, dcs=dcs1,
             dcd=dcd1, das=das1, dad=dad1, rows=rows1, sem_in=sem_in1,
             sem_g=sem_g1, sem_r=sem_r1, sem_o=sem_o1, sem_n=sem_n1),
    ]

    def fire_loads(off, S):
      return [
          pltpu.async_copy(src_hbm.at[pl.ds(off, CH_P)], S['ps'], S['sem_in']),
          pltpu.async_copy(dst_hbm.at[pl.ds(off, CH_P)], S['pd'], S['sem_in']),
          pltpu.async_copy(ew_hbm.at[pl.ds(off, CH_P)], S['wp'], S['sem_in']),
      ]

    def fire_gathers(S):
      g = [
          pltpu.async_copy(degc_sh.at[S['ps']], S['dcs'], S['sem_g']),
          pltpu.async_copy(degc_sh.at[S['pd']], S['dcd'], S['sem_g']),
          pltpu.async_copy(dega_sh.at[S['ps']], S['das'], S['sem_g']),
          pltpu.async_copy(dega_sh.at[S['pd']], S['dad'], S['sem_g']),
      ]
      r = [pltpu.async_copy(g_hbm.at[S['ps']], S['rows'], S['sem_r'])]
      return g, r

    def compute_and_out(off, S):
      for g in range(CH_P // 16):
        sl = pl.ds(g * 16, 16)
        sv = S['ps'][sl]
        dv = S['pd'][sl]
        wv = S['wp'][sl]
        wc = jnp.where(sv == dv, 0.0, wv)
        S['nc'][sl] = -(S['dcs'][sl] * wc * S['dcd'][sl])
        S['na'][sl] = S['das'][sl] * wv * S['dad'][sl]
      nw = [
          pltpu.async_copy(S['nc'], normc_hbm.at[pl.ds(off, CH_P)], S['sem_n']),
          pltpu.async_copy(S['na'], norma_hbm.at[pl.ds(off, CH_P)], S['sem_n']),
      ]
      return nw

    def fire_scatter(S):
      return [pltpu.async_copy(S['rows'], acc_sh.at[S['pdx']],
                               S['sem_o'], add=True)]

    def wait_prev_scatter(S, t):
      @pl.when(t > 0)
      def _():
        pltpu.make_async_copy(S['rows'], acc_sh.at[S['pdx']],
                              S['sem_o']).wait()

    def wait_prev_nw(S, t):
      @pl.when(t > 0)
      def _():
        pltpu.make_async_copy(S['nc'], normc_hbm.at[pl.ds(0, CH_P)],
                              S['sem_n']).wait()
        pltpu.make_async_copy(S['na'], norma_hbm.at[pl.ds(0, CH_P)],
                              S['sem_n']).wait()

    def snap_idx(S):
      for g in range(CH_P // 16):
        sl = pl.ds(g * 16, 16)
        S['pdx'][sl] = S['pd'][sl]

    @pl.loop(0, PASS_PAIRS)
    def _(t):
      off0 = pl.multiple_of(ebase + t * (2 * CH_P), CH_P)
      off1 = pl.multiple_of(ebase + t * (2 * CH_P) + CH_P, CH_P)
      S0, S1 = sets
      l0 = fire_loads(off0, S0)
      l1 = fire_loads(off1, S1)
      _drain(l0)
      wait_prev_scatter(S0, t)
      g0, r0 = fire_gathers(S0)
      _drain(l1)
      wait_prev_scatter(S1, t)
      g1, r1 = fire_gathers(S1)
      _drain(g0)
      wait_prev_nw(S0, t)
      nw0 = compute_and_out(off0, S0)
      _drain(r0)
      _scale_rows(S0['rows'], S0['nc'], S0['na'])
      snap_idx(S0)
      fire_scatter(S0)
      _drain(g1)
      wait_prev_nw(S1, t)
      nw1 = compute_and_out(off1, S1)
      _drain(r1)
      _scale_rows(S1['rows'], S1['nc'], S1['na'])
      snap_idx(S1)
      fire_scatter(S1)

    for S in sets:
      pltpu.make_async_copy(S['rows'], acc_sh.at[S['pdx']], S['sem_o']).wait()
      pltpu.make_async_copy(S['nc'], normc_hbm.at[pl.ds(0, CH_P)],
                            S['sem_n']).wait()
      pltpu.make_async_copy(S['na'], norma_hbm.at[pl.ds(0, CH_P)],
                            S['sem_n']).wait()
    plsc.subcore_barrier()

    @pl.when(s < NSUB - 1)
    def _():
      pltpu.sync_copy(acc_sh.at[pl.ds(rbase, DROWS), :],
                      p_hbm.at[c, pl.ds(rbase, DROWS), :])
    @pl.when(s == NSUB - 1)
    def _():
      pltpu.sync_copy(acc_sh.at[pl.ds(rbase, N - (NSUB - 1) * DROWS), :],
                      p_hbm.at[c, pl.ds(rbase, N - (NSUB - 1) * DROWS), :])

  sems = [pltpu.SemaphoreType.DMA] * 13
  return pl.kernel(
      body,
      out_type=(
          jax.ShapeDtypeStruct((NCORES, N, 128), jnp.float32),
          jax.ShapeDtypeStruct((EP,), jnp.float32),
          jax.ShapeDtypeStruct((EP,), jnp.float32),
      ),
      mesh=mesh,
      compiler_params=pltpu.CompilerParams(use_tc_tiling_on_sc=False),
      scratch_types=[
          pltpu.VMEM_SHARED((DN, 128), jnp.float32),
          pltpu.VMEM_SHARED((DN,), jnp.float32),
          pltpu.VMEM_SHARED((DN,), jnp.float32),
          pltpu.VMEM((16, 128), jnp.float32),
          pltpu.VMEM((DROWS,), jnp.float32),
          pltpu.VMEM((CH_D,), jnp.float32),
          pltpu.VMEM((CH_D,), jnp.float32),
          pltpu.VMEM((CH_D,), jnp.float32),
          pltpu.VMEM((CH_D,), jnp.float32),
      ] + [pltpu.VMEM((SUB,), jnp.int32)] * 16
        + [pltpu.VMEM((CH_P,), jnp.int32)] * 6
        + [pltpu.VMEM((CH_P,), jnp.float32)] * 14
        + [pltpu.VMEM((CH_P, 128), jnp.float32)] * 2
        + sems,
      name="sc_deg_norm_pass0",
  )


def _make_sc_pass2():
  mesh = plsc.VectorSubcoreMesh(
      core_axis_name="c", subcore_axis_name="s",
      num_cores=NCORES, num_subcores=NSUB)

  def body(src_hbm, dst_hbm, normc_hbm, norma_hbm, g_hbm, p_hbm,
           acc_sh, zrow,
           ps0, ps1, pd0, pd1, pdx0, pdx1, nc0, nc1, na0, na1,
           rows0, rows1,
           sem_z, sem_in0, sem_in1, sem_r0, sem_r1, sem_o0, sem_o1):
    c = lax.axis_index("c")
    s = lax.axis_index("s")

    _zero_acc(acc_sh, zrow, s, sem_z)
    plsc.subcore_barrier()

    ebase = (c * NSUB + s) * E_PASS
    sets = [
        dict(ps=ps0, pd=pd0, pdx=pdx0, nc=nc0, na=na0, rows=rows0,
             sem_in=sem_in0, sem_r=sem_r0, sem_o=sem_o0),
        dict(ps=ps1, pd=pd1, pdx=pdx1, nc=nc1, na=na1, rows=rows1,
             sem_in=sem_in1, sem_r=sem_r1, sem_o=sem_o1),
    ]

    def fire_loads(off, S):
      return [
          pltpu.async_copy(src_hbm.at[pl.ds(off, CH_P)], S['ps'], S['sem_in']),
          pltpu.async_copy(dst_hbm.at[pl.ds(off, CH_P)], S['pd'], S['sem_in']),
          pltpu.async_copy(normc_hbm.at[pl.ds(off, CH_P)], S['nc'],
                           S['sem_in']),
          pltpu.async_copy(norma_hbm.at[pl.ds(off, CH_P)], S['na'],
                           S['sem_in']),
      ]

    def wait_prev(S, t):
      @pl.when(t > 0)
      def _():
        pltpu.make_async_copy(S['rows'], acc_sh.at[S['pdx']],
                              S['sem_o']).wait()

    def snap_idx(S):
      for g in range(CH_P // 16):
        sl = pl.ds(g * 16, 16)
        S['pdx'][sl] = S['pd'][sl]

    @pl.loop(0, PASS_PAIRS)
    def _(t):
      off0 = pl.multiple_of(ebase + t * (2 * CH_P), CH_P)
      off1 = pl.multiple_of(ebase + t * (2 * CH_P) + CH_P, CH_P)
      S0, S1 = sets
      l0 = fire_loads(off0, S0)
      l1 = fire_loads(off1, S1)
      _drain(l0)
      wait_prev(S0, t)
      r0 = [pltpu.async_copy(g_hbm.at[S0['ps']], S0['rows'], S0['sem_r'])]
      _drain(l1)
      wait_prev(S1, t)
      r1 = [pltpu.async_copy(g_hbm.at[S1['ps']], S1['rows'], S1['sem_r'])]
      _drain(r0)
      _scale_rows(S0['rows'], S0['nc'], S0['na'])
      snap_idx(S0)
      pltpu.async_copy(S0['rows'], acc_sh.at[S0['pdx']], S0['sem_o'], add=True)
      _drain(r1)
      _scale_rows(S1['rows'], S1['nc'], S1['na'])
      snap_idx(S1)
      pltpu.async_copy(S1['rows'], acc_sh.at[S1['pdx']], S1['sem_o'], add=True)

    for S in sets:
      pltpu.make_async_copy(S['rows'], acc_sh.at[S['pdx']], S['sem_o']).wait()
    plsc.subcore_barrier()
    rbase = s * DROWS
    @pl.when(s < NSUB - 1)
    def _():
      pltpu.sync_copy(acc_sh.at[pl.ds(rbase, DROWS), :],
                      p_hbm.at[c, pl.ds(rbase, DROWS), :])
    @pl.when(s == NSUB - 1)
    def _():
      pltpu.sync_copy(acc_sh.at[pl.ds(rbase, N - (NSUB - 1) * DROWS), :],
                      p_hbm.at[c, pl.ds(rbase, N - (NSUB - 1) * DROWS), :])

  return pl.kernel(
      body,
      out_type=jax.ShapeDtypeStruct((NCORES, N, 128), jnp.float32),
      mesh=mesh,
      compiler_params=pltpu.CompilerParams(use_tc_tiling_on_sc=False),
      scratch_types=[
          pltpu.VMEM_SHARED((DN, 128), jnp.float32),
          pltpu.VMEM((16, 128), jnp.float32),
      ] + [pltpu.VMEM((CH_P,), jnp.int32)] * 6
        + [pltpu.VMEM((CH_P,), jnp.float32)] * 4
        + [pltpu.VMEM((CH_P, 128), jnp.float32)] * 2
        + [pltpu.SemaphoreType.DMA] * 7,
      name="sc_pass1",
  )


_sc_pass1 = _make_sc_pass1()
_sc_pass2 = _make_sc_pass2()


RB = 1000
GRID = N // RB


def _dotT(a, w):
  return lax.dot_general(a, w, (((1,), (1,)), ((), ())),
                         preferred_element_type=jnp.float32)


def _dot(a, w):
  return lax.dot_general(a, w, (((1,), (0,)), ((), ())),
                         preferred_element_type=jnp.float32)


def _tc1_body(x_r, pxW_r, pxb_r, phW_r, phb_r, aiW_r, g_r, xp_r):
  xb = x_r[...]
  xp = _dotT(xb, pxW_r[...]) + pxb_r[...]
  hp = _dotT(xb, phW_r[...]) + phb_r[...]
  hh = _dot(xp, aiW_r[...])
  g_r[...] = jnp.concatenate([hp, hh], axis=1)
  xp_r[...] = xp


def _tc2_body(x_r, g0_r, xp0_r, p_r,
              cl0_r, cl1_r, cb_r, arW_r, ab_r, lW_r, lb_r,
              pxW_r, pxb_r, phW_r, phb_r, aiW_r,
              g1_r, xp1_r):
  p = p_r[0] + p_r[1]
  tx1 = p[:, :64]
  agg = p[:, 64:]
  hp0 = g0_r[:, :64]
  xp0 = xp0_r[...]
  o1 = _dotT(hp0, cl0_r[...]) + _dotT(tx1, cl1_r[...]) + cb_r[...]
  o1 = jnp.where(o1 >= 0, o1, 0.01 * o1)
  o2 = agg + _dot(xp0, arW_r[...]) + ab_r[...]
  o2 = jnp.maximum(o2, 0.0)
  o3 = _dotT(o1 + o2, lW_r[...]) + lb_r[...]
  xp1 = _dotT(o3, pxW_r[...]) + pxb_r[...]
  hp1 = _dotT(x_r[...], phW_r[...]) + phb_r[...]
  hh1 = _dot(xp1, aiW_r[...])
  g1_r[...] = jnp.concatenate([hp1, hh1], axis=1)
  xp1_r[...] = xp1


def _tc3_body(g1_r, xp1_r, p_r,
              cl0_r, cl1_r, cb_r, arW_r, ab_r, lW_r, lb_r,
              clsW_r, clsb_r, out_r):
  p = p_r[0] + p_r[1]
  tx1 = p[:, :64]
  agg = p[:, 64:]
  hp1 = g1_r[:, :64]
  o1 = _dotT(hp1, cl0_r[...]) + _dotT(tx1, cl1_r[...]) + cb_r[...]
  o1 = jnp.where(o1 >= 0, o1, 0.01 * o1)
  o2 = agg + _dot(xp1_r[...], arW_r[...]) + ab_r[...]
  o2 = jnp.maximum(o2, 0.0)
  o3 = _dotT(o1 + o2, lW_r[...]) + lb_r[...]
  logits = _dotT(o3, clsW_r[...]) + clsb_r[...]
  m = jnp.max(logits, axis=1, keepdims=True)
  sh = logits - m
  out_r[...] = sh - jnp.log(jnp.sum(jnp.exp(sh), axis=1, keepdims=True))


def _full(shape):
  return pl.BlockSpec(shape, lambda i: (0,) * len(shape))


def _rows(shape):
  return pl.BlockSpec(shape, lambda i: (i,) + (0,) * (len(shape) - 1))


def kernel(x, edge_index, edge_weight,
           c0_pre_h_W, c0_pre_h_b, c0_pre_x_W, c0_pre_x_b,
           c0_cheb_lin0_W, c0_cheb_lin1_W, c0_cheb_b,
           c0_arma_init_W, c0_arma_root_W, c0_arma_b,
           c0_lin_W, c0_lin_b,
           c1_pre_h_W, c1_pre_h_b, c1_pre_x_W, c1_pre_x_b,
           c1_cheb_lin0_W, c1_cheb_lin1_W, c1_cheb_b,
           c1_arma_init_W, c1_arma_root_W, c1_arma_b,
           c1_lin_W, c1_lin_b,
           cls_W, cls_b):
  r2 = lambda b: b.reshape(1, -1)

  pad_idx = (jnp.arange(PAD, dtype=jnp.int32) * 997) % N
  src = jnp.concatenate([edge_index[0], pad_idx])
  dst = jnp.concatenate([edge_index[1], pad_idx])
  ew = jnp.concatenate([edge_weight, jnp.zeros((PAD,), jnp.float32)])

  g0, xp0 = pl.pallas_call(
      _tc1_body,
      grid=(GRID,),
      in_specs=[_rows((RB, 128)), _full((64, 128)), _full((1, 64)),
                _full((64, 128)), _full((1, 64)), _full((64, 64))],
      out_specs=[_rows((RB, 128)), _rows((RB, 64))],
      out_shape=[jax.ShapeDtypeStruct((N, 128), jnp.float32),
                 jax.ShapeDtypeStruct((N, 64), jnp.float32)],
  )(x, c0_pre_x_W, r2(c0_pre_x_b), c0_pre_h_W, r2(c0_pre_h_b),
    c0_arma_init_W)

  p0, normc, norma = _sc_pass1(src, dst, ew, g0)

  g1, xp1 = pl.pallas_call(
      _tc2_body,
      grid=(GRID,),
      in_specs=[_rows((RB, 128)), _rows((RB, 128)), _rows((RB, 64)),
                pl.BlockSpec((2, RB, 128), lambda i: (0, i, 0)),
                _full((64, 64)), _full((64, 64)), _full((1, 64)),
                _full((64, 64)), _full((1, 64)),
                _full((64, 64)), _full((1, 64)),
                _full((64, 64)), _full((1, 64)),
                _full((64, 128)), _full((1, 64)), _full((64, 64))],
      out_specs=[_rows((RB, 128)), _rows((RB, 64))],
      out_shape=[jax.ShapeDtypeStruct((N, 128), jnp.float32),
                 jax.ShapeDtypeStruct((N, 64), jnp.float32)],
  )(x, g0, xp0, p0,
    c0_cheb_lin0_W, c0_cheb_lin1_W, r2(c0_cheb_b),
    c0_arma_root_W, r2(c0_arma_b), c0_lin_W, r2(c0_lin_b),
    c1_pre_x_W, r2(c1_pre_x_b), c1_pre_h_W, r2(c1_pre_h_b),
    c1_arma_init_W)

  p1 = _sc_pass2(src, dst, normc, norma, g1)

  out = pl.pallas_call(
      _tc3_body,
      grid=(GRID,),
      in_specs=[_rows((RB, 128)), _rows((RB, 64)),
                pl.BlockSpec((2, RB, 128), lambda i: (0, i, 0)),
                _full((64, 64)), _full((64, 64)), _full((1, 64)),
                _full((64, 64)), _full((1, 64)),
                _full((64, 64)), _full((1, 64)),
                _full((32, 64)), _full((1, 32))],
      out_specs=_rows((RB, NC_CLS)),
      out_shape=jax.ShapeDtypeStruct((N, NC_CLS), jnp.float32),
  )(g1, xp1, p1,
    c1_cheb_lin0_W, c1_cheb_lin1_W, r2(c1_cheb_b),
    c1_arma_root_W, r2(c1_arma_b), c1_lin_W, r2(c1_lin_b),
    cls_W, r2(cls_b))

  return out
